# Initial kernel scaffold; baseline (speedup 1.0000x reference)
import jax
import jax.numpy as jnp
from jax.experimental import pallas as pl


def kernel(person_x, object_x, location_x, acts_edge, uses_src, uses_dst, at_src, at_dst, person_table, object_table, location_table, conv_Wn, conv_Wr, conv_b, cls_W1, cls_b1, cls_W2, cls_b2, sp_W1, sp_b1, sp_W2, sp_b2):
    raise NotImplementedError("write your pallas kernel here")


# speedup vs baseline: 4.8266x; 4.8266x over previous
import functools

import jax
import jax.numpy as jnp
from jax import lax
from jax.experimental import pallas as pl
from jax.experimental.pallas import tpu as pltpu
from jax.experimental.pallas import tpu_sc as plsc

_HD = 64
_NP, _NO, _NL = 50000, 20000, 10000
_E = 800000
_NC = 2
_NS = 16
_K = 4
_CH = _K * 128
_PT = 50176
_EPAD = _NS * _PT
_ROWS_PER_TILE = _PT // 128
_NMACRO = _ROWS_PER_TILE // _K


def _ndpad(n):
    return -(-(n + 16) // 256) * 256


@functools.lru_cache(None)
def _seg_kernel(n_src, nd_pad):
    zr = nd_pad // 16
    mesh = plsc.VectorSubcoreMesh(core_axis_name="c", subcore_axis_name="s")

    @functools.partial(
        pl.kernel,
        mesh=mesh,
        compiler_params=pltpu.CompilerParams(use_tc_tiling_on_sc=False),
        out_type=jax.ShapeDtypeStruct((2 * nd_pad, 32), jnp.float32),
        scratch_types=[
            pltpu.VMEM((_K, 128), jnp.int32),
            pltpu.VMEM((_K, 128), jnp.int32),
            pltpu.VMEM((_CH, 32), jnp.float32),
            pltpu.VMEM_SHARED((nd_pad, 32), jnp.float32),
            pltpu.SemaphoreType.DMA,
        ],
    )
    def seg(z_hbm, src_hbm, dst_hbm, zero_hbm, out_hbm, srcv, dstv, rows, acc, sem):
        c = lax.axis_index("c")
        s = lax.axis_index("s")
        r0 = s * zr
        pltpu.sync_copy(zero_hbm.at[pl.ds(r0, zr)], acc.at[pl.ds(r0, zr)])
        plsc.subcore_barrier()

        def body(m, carry):
            rb = s * _ROWS_PER_TILE + m * _K
            pltpu.sync_copy(src_hbm.at[pl.ds(rb, _K)], srcv)
            pltpu.sync_copy(dst_hbm.at[pl.ds(rb, _K)], dstv)
            off = c * n_src
            for j in range(_K):
                for t in range(8):
                    srcv[j, pl.ds(t * 16, 16)] = srcv[j, pl.ds(t * 16, 16)] + off
            cops = [
                pltpu.async_copy(z_hbm.at[srcv.at[j]], rows.at[pl.ds(j * 128, 128)], sem)
                for j in range(_K)
            ]
            for cp in cops:
                cp.wait()
            for j in range(_K):
                pltpu.sync_copy(rows.at[pl.ds(j * 128, 128)], acc.at[dstv.at[j]], add=True)
            return carry

        lax.fori_loop(0, _NMACRO, body, 0)
        plsc.subcore_barrier()
        pltpu.sync_copy(acc.at[pl.ds(r0, zr)], out_hbm.at[pl.ds(c * nd_pad + r0, zr)])

    return seg


def _seg(z2, src_rows, dst_rows, n_src, n_dst):
    nd_pad = _ndpad(n_dst)
    zeros = jnp.zeros((nd_pad, 32), jnp.float32)
    return _seg_kernel(n_src, nd_pad)(z2, src_rows, dst_rows, zeros)


@functools.lru_cache(None)
def _embed_kernel(b_pad, wb, wk):
    mesh = plsc.VectorSubcoreMesh(core_axis_name="c", subcore_axis_name="s")

    @functools.partial(
        pl.kernel,
        mesh=mesh,
        compiler_params=pltpu.CompilerParams(use_tc_tiling_on_sc=False),
        out_type=jax.ShapeDtypeStruct((b_pad, _HD), jnp.float32),
        scratch_types=[
            pltpu.VMEM((wk, 128), jnp.int32),
            pltpu.VMEM((wb, _HD), jnp.float32),
            pltpu.SemaphoreType.DMA,
        ],
    )
    def emb(tab_hbm, idx_hbm, out_hbm, idxv, rows, sem):
        c = lax.axis_index("c")
        s = lax.axis_index("s")
        w = s * _NC + c
        pltpu.sync_copy(idx_hbm.at[w], idxv)
        cops = [
            pltpu.async_copy(tab_hbm.at[idxv.at[j]], rows.at[pl.ds(j * 128, 128)], sem)
            for j in range(wk)
        ]
        for cp in cops:
            cp.wait()
        pltpu.sync_copy(rows, out_hbm.at[pl.ds(w * wb, wb)])

    return emb


def _embed(table, idx, n):
    nw = _NC * _NS
    wb = -(-n // (nw * 128)) * 128
    b_pad = nw * wb
    wk = wb // 128
    pad = b_pad - n
    idx_p = jnp.concatenate(
        [idx.astype(jnp.int32), jnp.arange(pad, dtype=jnp.int32) % table.shape[0]]
    ).reshape(nw, wk, 128)
    out = _embed_kernel(b_pad, wb, wk)(table, idx_p)
    return out[:n]


def _pad_edges(src, dst, n_src, n_dst):
    pad = _EPAD - _E
    ar = jnp.arange(pad, dtype=jnp.int32)
    src_p = jnp.concatenate([src.astype(jnp.int32), ar % n_src])
    dst_p = jnp.concatenate([dst.astype(jnp.int32), n_dst + (ar % 16)])
    return (src_p.reshape(_EPAD // 128, 128), dst_p.reshape(_EPAD // 128, 128),
            n_src, n_dst)


_BN = 2000


def _mm_split(x, ws):
    n = x.shape[0]
    g = n // _BN
    nw = len(ws)

    def kern(*refs):
        x_ref = refs[0]
        w_refs = refs[1:1 + nw]
        o_refs = refs[1 + nw:]
        xb = x_ref[...]
        for wr, orf in zip(w_refs, o_refs):
            z = jnp.dot(xb, wr[...], preferred_element_type=jnp.float32)
            orf[0] = z[:, :32]
            orf[1] = z[:, 32:64]

    return pl.pallas_call(
        kern,
        grid=(g,),
        in_specs=[pl.BlockSpec((_BN, _HD), lambda i: (i, 0))]
        + [pl.BlockSpec((_HD, _HD), lambda i: (0, 0))] * nw,
        out_specs=[pl.BlockSpec((2, _BN, 32), lambda i: (0, i, 0))] * nw,
        out_shape=[jax.ShapeDtypeStruct((2, n, 32), jnp.float32)] * nw,
    )(x, *ws)


def _combine(accs, cnts, x, w, b, inv_nrel):
    n = x.shape[0]
    g = n // _BN
    nr = len(accs)

    def kern(*refs):
        a_refs = refs[:nr]
        c_refs = refs[nr:2 * nr]
        xr, wr, br = refs[2 * nr:2 * nr + 3]
        orf = refs[2 * nr + 3]
        m = None
        for ar, cr in zip(a_refs, c_refs):
            feat = jnp.concatenate([ar[0], ar[1]], axis=1)
            cnt = jnp.maximum(cr[0, :, :1], 1.0)
            t = feat / cnt
            m = t if m is None else m + t
        pn = m * inv_nrel + jnp.dot(xr[...], wr[...],
                                    preferred_element_type=jnp.float32) + br[...]
        orf[...] = jnp.maximum(pn, 0.0) + xr[...]

    nd_pad = accs[0].shape[0] // 2
    a3 = [a.reshape(2, nd_pad, 32) for a in accs]
    c3 = [c.reshape(2, nd_pad, 32) for c in cnts]
    return pl.pallas_call(
        kern,
        grid=(g,),
        in_specs=(
            [pl.BlockSpec((2, _BN, 32), lambda i: (0, i, 0))] * nr
            + [pl.BlockSpec((1, _BN, 32), lambda i: (0, i, 0))] * nr
            + [pl.BlockSpec((_BN, _HD), lambda i: (i, 0)),
               pl.BlockSpec((_HD, _HD), lambda i: (0, 0)),
               pl.BlockSpec((1, _HD), lambda i: (0, 0))]
        ),
        out_specs=pl.BlockSpec((_BN, _HD), lambda i: (i, 0)),
        out_shape=jax.ShapeDtypeStruct((n, _HD), jnp.float32),
    )(*a3, *c3, x, w, b.reshape(1, _HD))


def _colsum(x):
    n = x.shape[0]
    g = n // _BN

    def kern(xr, orf):
        @pl.when(pl.program_id(0) == 0)
        def _():
            orf[...] = jnp.zeros_like(orf)

        orf[...] += jnp.sum(xr[...], axis=0, keepdims=True)

    return pl.pallas_call(
        kern,
        grid=(g,),
        in_specs=[pl.BlockSpec((_BN, _HD), lambda i: (i, 0))],
        out_specs=pl.BlockSpec((1, _HD), lambda i: (0, 0)),
        out_shape=jax.ShapeDtypeStruct((1, _HD), jnp.float32),
    )(x)


def _cls_head(sums, scale, w1, b1, w2, b2):
    def kern(sr, scr, w1r, b1r, w2r, b2r, oge, olg):
        ge = sr[...] * scr[...]
        oge[...] = ge
        h = jnp.maximum(
            jnp.dot(ge, w1r[...], preferred_element_type=jnp.float32) + b1r[...], 0.0)
        olg[...] = jnp.dot(h, w2r[...], preferred_element_type=jnp.float32) + b2r[...]

    return pl.pallas_call(
        kern,
        out_shape=[jax.ShapeDtypeStruct((1, 3 * _HD), jnp.float32),
                   jax.ShapeDtypeStruct((1, 50), jnp.float32)],
    )(sums, scale, w1, b1.reshape(1, _HD), w2, b2.reshape(1, 50))


def _scores(x, w1, b1, w2t, b2):
    n = x.shape[0]
    g = n // _BN

    def kern(xr, w1r, b1r, w2r, b2r, orf):
        h = jnp.maximum(
            jnp.dot(xr[...], w1r[...], preferred_element_type=jnp.float32) + b1r[...],
            0.0)
        sv = jnp.sum(h * w2r[...], axis=1, keepdims=True)
        orf[...] = sv + b2r[...]

    out = pl.pallas_call(
        kern,
        grid=(g,),
        in_specs=[pl.BlockSpec((_BN, _HD), lambda i: (i, 0)),
                  pl.BlockSpec((_HD, 32), lambda i: (0, 0)),
                  pl.BlockSpec((1, 32), lambda i: (0, 0)),
                  pl.BlockSpec((1, 32), lambda i: (0, 0)),
                  pl.BlockSpec((1, 128), lambda i: (0, 0))],
        out_specs=pl.BlockSpec((_BN, 128), lambda i: (i, 0)),
        out_shape=jax.ShapeDtypeStruct((n, 128), jnp.float32),
    )(x, w1, b1.reshape(1, 32), w2t, b2)
    return out[:, 0]


def kernel(person_x, object_x, location_x, acts_edge, uses_src, uses_dst, at_src,
           at_dst, person_table, object_table, location_table, conv_Wn, conv_Wr,
           conv_b, cls_W1, cls_b1, cls_W2, cls_b2, sp_W1, sp_b1, sp_W2, sp_b2):
    f32 = jnp.float32
    xp = _embed(person_table, person_x, _NP)
    xo = _embed(object_table, object_x, _NO)
    xl = _embed(location_table, location_x, _NL)

    e0 = _pad_edges(acts_edge[0], acts_edge[1], _NP, _NP)
    e1 = _pad_edges(uses_src, uses_dst, _NP, _NO)
    e2 = _pad_edges(uses_dst, uses_src, _NO, _NP)
    e3 = _pad_edges(at_src, at_dst, _NP, _NL)
    e4 = _pad_edges(at_dst, at_src, _NL, _NP)

    ones_p = jnp.ones((2 * _NP, 32), f32)
    ones_o = jnp.ones((2 * _NO, 32), f32)
    ones_l = jnp.ones((2 * _NL, 32), f32)
    c0 = _seg(ones_p, *e0)
    c1 = _seg(ones_p, *e1)
    c2 = _seg(ones_o, *e2)
    c3 = _seg(ones_p, *e3)
    c4 = _seg(ones_l, *e4)

    wr_p = (conv_Wr[:, 0] + conv_Wr[:, 2] + conv_Wr[:, 4]) / 3.0
    b_p = (conv_b[:, 0] + conv_b[:, 2] + conv_b[:, 4]) / 3.0

    for l in range(3):
        z0, z1, z3 = _mm_split(xp, [conv_Wn[l, 0], conv_Wn[l, 1], conv_Wn[l, 3]])
        (z2,) = _mm_split(xo, [conv_Wn[l, 2]])
        (z4,) = _mm_split(xl, [conv_Wn[l, 4]])
        s0 = _seg(z0.reshape(2 * _NP, 32), *e0)
        s1 = _seg(z1.reshape(2 * _NP, 32), *e1)
        s2 = _seg(z2.reshape(2 * _NO, 32), *e2)
        s3 = _seg(z3.reshape(2 * _NP, 32), *e3)
        s4 = _seg(z4.reshape(2 * _NL, 32), *e4)
        xp = _combine([s0, s2, s4], [c0, c2, c4], xp, wr_p[l], b_p[l], 1.0 / 3.0)
        xo = _combine([s1], [c1], xo, conv_Wr[l, 1], conv_b[l, 1], 1.0)
        xl = _combine([s3], [c3], xl, conv_Wr[l, 3], conv_b[l, 3], 1.0)

    sums = jnp.concatenate([_colsum(xp), _colsum(xo), _colsum(xl)], axis=1)
    scale = jnp.concatenate(
        [jnp.full((1, _HD), 1.0 / _NP, f32), jnp.full((1, _HD), 1.0 / _NO, f32),
         jnp.full((1, _HD), 1.0 / _NL, f32)], axis=1)
    ge, logits = _cls_head(sums, scale, cls_W1, cls_b1, cls_W2, cls_b2)

    b2full = jnp.full((1, 128), sp_b2[0], f32)
    scores = _scores(xp, sp_W1, sp_b1, sp_W2.reshape(1, 32), b2full)

    return logits, scores, ge.reshape(3 * _HD)


# speedup vs baseline: 8.2398x; 1.7071x over previous
import functools

import jax
import jax.numpy as jnp
from jax import lax
from jax.experimental import pallas as pl
from jax.experimental.pallas import tpu as pltpu
from jax.experimental.pallas import tpu_sc as plsc

_HD = 64
_NP, _NO, _NL = 50000, 20000, 10000
_E = 800000
_NC = 2
_NS = 16
_K = 4
_CH = _K * 128
_PT = 50176
_EPAD = _NS * _PT + _K * 128
_ROWS_PER_TILE = _PT // 128
_NMACRO = _ROWS_PER_TILE // _K


def _ndpad(n):
    return -(-(n + 16) // 256) * 256


@functools.lru_cache(None)
def _seg_kernel(n_src, nd_pad):
    zr = nd_pad // 16
    mesh = plsc.VectorSubcoreMesh(core_axis_name="c", subcore_axis_name="s")

    @functools.partial(
        pl.kernel,
        mesh=mesh,
        compiler_params=pltpu.CompilerParams(use_tc_tiling_on_sc=False),
        out_type=jax.ShapeDtypeStruct((2 * nd_pad, 32), jnp.float32),
        scratch_types=[
            pltpu.VMEM((_K, 128), jnp.int32),
            pltpu.VMEM((_K, 128), jnp.int32),
            pltpu.VMEM((_K, 128), jnp.int32),
            pltpu.VMEM((_K, 128), jnp.int32),
            pltpu.VMEM((_CH, 32), jnp.float32),
            pltpu.VMEM_SHARED((nd_pad, 32), jnp.float32),
            pltpu.SemaphoreType.DMA,
        ],
    )
    def seg(z_hbm, src_hbm, dst_hbm, zero_hbm, out_hbm,
            src_a, dst_a, src_b, dst_b, rows, acc, sem):
        c = lax.axis_index("c")
        s = lax.axis_index("s")
        off = c * n_src
        r0 = s * zr
        pltpu.sync_copy(zero_hbm.at[pl.ds(r0, zr)], acc.at[pl.ds(r0, zr)])
        plsc.subcore_barrier()

        def load_idx(m, srcv, dstv):
            rb = s * _ROWS_PER_TILE + m * _K
            pltpu.sync_copy(src_hbm.at[pl.ds(rb, _K)], srcv)
            pltpu.sync_copy(dst_hbm.at[pl.ds(rb, _K)], dstv)
            for j in range(_K):
                for t in range(8):
                    srcv[j, pl.ds(t * 16, 16)] = srcv[j, pl.ds(t * 16, 16)] + off

        def fire(srcv, j):
            pltpu.async_copy(z_hbm.at[srcv.at[j]], rows.at[pl.ds(j * 128, 128)], sem)

        def drain(j):
            pltpu.make_async_copy(
                z_hbm.at[pl.ds(0, 128)], rows.at[pl.ds(j * 128, 128)], sem).wait()

        def chunk(m, cur_s, cur_d, nxt_s, nxt_d):
            load_idx(m + 1, nxt_s, nxt_d)
            for j in range(_K):
                drain(j)
            for j in range(_K):
                pltpu.sync_copy(rows.at[pl.ds(j * 128, 128)],
                                acc.at[cur_d.at[j]], add=True)
                fire(nxt_s, j)

        load_idx(0, src_a, dst_a)
        for j in range(_K):
            fire(src_a, j)

        def body(i, carry):
            chunk(2 * i, src_a, dst_a, src_b, dst_b)
            chunk(2 * i + 1, src_b, dst_b, src_a, dst_a)
            return carry

        lax.fori_loop(0, _NMACRO // 2, body, 0)
        for j in range(_K):
            drain(j)
        plsc.subcore_barrier()
        pltpu.sync_copy(acc.at[pl.ds(r0, zr)], out_hbm.at[pl.ds(c * nd_pad + r0, zr)])

    return seg


def _seg(z2, src_rows, dst_rows, n_src, n_dst):
    nd_pad = _ndpad(n_dst)
    zeros = jnp.zeros((nd_pad, 32), jnp.float32)
    return _seg_kernel(n_src, nd_pad)(z2, src_rows, dst_rows, zeros)


_CNT_NDS = (_ndpad(_NP), _ndpad(_NO), _ndpad(_NP), _ndpad(_NL), _ndpad(_NP))


@functools.lru_cache(None)
def _counts_kernel():
    mesh = plsc.VectorSubcoreMesh(core_axis_name="c", subcore_axis_name="s")
    half = _ROWS_PER_TILE // 2
    nch = half // _K
    nd_max = max(_CNT_NDS)

    @functools.partial(
        pl.kernel,
        mesh=mesh,
        compiler_params=pltpu.CompilerParams(use_tc_tiling_on_sc=False),
        out_type=[jax.ShapeDtypeStruct((2 * nd, 16), jnp.float32)
                  for nd in _CNT_NDS],
        scratch_types=[
            pltpu.VMEM((_K, 128), jnp.int32),
            pltpu.VMEM((128, 16), jnp.float32),
            pltpu.VMEM_SHARED((nd_max, 16), jnp.float32),
        ],
    )
    def cnt(d0, d1, d2, d3, d4, zero_hbm, o0, o1, o2, o3, o4, dstv, ones, cacc):
        c = lax.axis_index("c")
        s = lax.axis_index("s")
        for i in range(128):
            ones[i, pl.ds(0, 16)] = jnp.full((16,), 1.0, jnp.float32)
        for dh, oh, nd in zip((d0, d1, d2, d3, d4), (o0, o1, o2, o3, o4),
                              _CNT_NDS):
            zrr = nd // 16
            r0 = s * zrr
            pltpu.sync_copy(zero_hbm.at[pl.ds(r0, zrr)], cacc.at[pl.ds(r0, zrr)])
            plsc.subcore_barrier()

            def body(m, carry, dh=dh):
                rb = s * _ROWS_PER_TILE + c * half + m * _K
                pltpu.sync_copy(dh.at[pl.ds(rb, _K)], dstv)
                for j in range(_K):
                    pltpu.sync_copy(ones, cacc.at[dstv.at[j]], add=True)
                return carry

            lax.fori_loop(0, nch, body, 0)
            plsc.subcore_barrier()
            pltpu.sync_copy(cacc.at[pl.ds(r0, zrr)],
                            oh.at[pl.ds(c * nd + r0, zrr)])
            plsc.subcore_barrier()

    return cnt


def _counts(edges):
    zeros = jnp.zeros((max(_CNT_NDS), 16), jnp.float32)
    outs = _counts_kernel()(*[e[1] for e in edges], zeros)
    return [o.reshape(2, nd, 16) for o, nd in zip(outs, _CNT_NDS)]


@functools.lru_cache(None)
def _embed_kernel(b_pad, wb, wk):
    mesh = plsc.VectorSubcoreMesh(core_axis_name="c", subcore_axis_name="s")

    @functools.partial(
        pl.kernel,
        mesh=mesh,
        compiler_params=pltpu.CompilerParams(use_tc_tiling_on_sc=False),
        out_type=jax.ShapeDtypeStruct((b_pad, _HD), jnp.float32),
        scratch_types=[
            pltpu.VMEM((wk, 128), jnp.int32),
            pltpu.VMEM((wb, _HD), jnp.float32),
            pltpu.SemaphoreType.DMA,
        ],
    )
    def emb(tab_hbm, idx_hbm, out_hbm, idxv, rows, sem):
        c = lax.axis_index("c")
        s = lax.axis_index("s")
        w = s * _NC + c
        pltpu.sync_copy(idx_hbm.at[w], idxv)
        cops = [
            pltpu.async_copy(tab_hbm.at[idxv.at[j]], rows.at[pl.ds(j * 128, 128)], sem)
            for j in range(wk)
        ]
        for cp in cops:
            cp.wait()
        pltpu.sync_copy(rows, out_hbm.at[pl.ds(w * wb, wb)])

    return emb


def _embed(table, idx, n):
    nw = _NC * _NS
    wb = -(-n // (nw * 128)) * 128
    b_pad = nw * wb
    wk = wb // 128
    pad = b_pad - n
    idx_p = jnp.concatenate(
        [idx.astype(jnp.int32), jnp.arange(pad, dtype=jnp.int32) % table.shape[0]]
    ).reshape(nw, wk, 128)
    out = _embed_kernel(b_pad, wb, wk)(table, idx_p)
    return out[:n]


def _pad_edges(src, dst, n_src, n_dst):
    pad = _EPAD - _E
    ar = jnp.arange(pad, dtype=jnp.int32)
    src_p = jnp.concatenate([src.astype(jnp.int32), ar % n_src])
    dst_p = jnp.concatenate([dst.astype(jnp.int32), n_dst + (ar % 16)])
    return (src_p.reshape(_EPAD // 128, 128), dst_p.reshape(_EPAD // 128, 128),
            n_src, n_dst)


_BN = 2000


def _mm_split(x, ws):
    n = x.shape[0]
    g = n // _BN
    nw = len(ws)

    def kern(*refs):
        x_ref = refs[0]
        w_refs = refs[1:1 + nw]
        o_refs = refs[1 + nw:]
        xb = x_ref[...]
        for wr, orf in zip(w_refs, o_refs):
            z = jnp.dot(xb, wr[...], preferred_element_type=jnp.float32)
            orf[0] = z[:, :32]
            orf[1] = z[:, 32:64]

    return pl.pallas_call(
        kern,
        grid=(g,),
        in_specs=[pl.BlockSpec((_BN, _HD), lambda i: (i, 0))]
        + [pl.BlockSpec((_HD, _HD), lambda i: (0, 0))] * nw,
        out_specs=[pl.BlockSpec((2, _BN, 32), lambda i: (0, i, 0))] * nw,
        out_shape=[jax.ShapeDtypeStruct((2, n, 32), jnp.float32)] * nw,
    )(x, *ws)


def _combine(accs, cnts, x, w, b, inv_nrel):
    n = x.shape[0]
    g = n // _BN
    nr = len(accs)

    def kern(*refs):
        a_refs = refs[:nr]
        c_refs = refs[nr:2 * nr]
        xr, wr, br = refs[2 * nr:2 * nr + 3]
        orf = refs[2 * nr + 3]
        m = None
        for ar, cr in zip(a_refs, c_refs):
            feat = jnp.concatenate([ar[0], ar[1]], axis=1)
            cnt = jnp.maximum(cr[0, :, :1] + cr[1, :, :1], 1.0)
            t = feat / cnt
            m = t if m is None else m + t
        pn = m * inv_nrel + jnp.dot(xr[...], wr[...],
                                    preferred_element_type=jnp.float32) + br[...]
        orf[...] = jnp.maximum(pn, 0.0) + xr[...]

    nd_pad = accs[0].shape[0] // 2
    a3 = [a.reshape(2, nd_pad, 32) for a in accs]
    c3 = list(cnts)
    return pl.pallas_call(
        kern,
        grid=(g,),
        in_specs=(
            [pl.BlockSpec((2, _BN, 32), lambda i: (0, i, 0))] * nr
            + [pl.BlockSpec((2, _BN, 16), lambda i: (0, i, 0))] * nr
            + [pl.BlockSpec((_BN, _HD), lambda i: (i, 0)),
               pl.BlockSpec((_HD, _HD), lambda i: (0, 0)),
               pl.BlockSpec((1, _HD), lambda i: (0, 0))]
        ),
        out_specs=pl.BlockSpec((_BN, _HD), lambda i: (i, 0)),
        out_shape=jax.ShapeDtypeStruct((n, _HD), jnp.float32),
    )(*a3, *c3, x, w, b.reshape(1, _HD))


def _colsum(x):
    n = x.shape[0]
    g = n // _BN

    def kern(xr, orf):
        @pl.when(pl.program_id(0) == 0)
        def _():
            orf[...] = jnp.zeros_like(orf)

        orf[...] += jnp.sum(xr[...], axis=0, keepdims=True)

    return pl.pallas_call(
        kern,
        grid=(g,),
        in_specs=[pl.BlockSpec((_BN, _HD), lambda i: (i, 0))],
        out_specs=pl.BlockSpec((1, _HD), lambda i: (0, 0)),
        out_shape=jax.ShapeDtypeStruct((1, _HD), jnp.float32),
    )(x)


def _cls_head(sums, scale, w1, b1, w2, b2):
    def kern(sr, scr, w1r, b1r, w2r, b2r, oge, olg):
        ge = sr[...] * scr[...]
        oge[...] = ge
        h = jnp.maximum(
            jnp.dot(ge, w1r[...], preferred_element_type=jnp.float32) + b1r[...], 0.0)
        olg[...] = jnp.dot(h, w2r[...], preferred_element_type=jnp.float32) + b2r[...]

    return pl.pallas_call(
        kern,
        out_shape=[jax.ShapeDtypeStruct((1, 3 * _HD), jnp.float32),
                   jax.ShapeDtypeStruct((1, 50), jnp.float32)],
    )(sums, scale, w1, b1.reshape(1, _HD), w2, b2.reshape(1, 50))


def _scores(x, w1, b1, w2t, b2):
    n = x.shape[0]
    g = n // _BN

    def kern(xr, w1r, b1r, w2r, b2r, orf):
        h = jnp.maximum(
            jnp.dot(xr[...], w1r[...], preferred_element_type=jnp.float32) + b1r[...],
            0.0)
        sv = jnp.sum(h * w2r[...], axis=1, keepdims=True)
        orf[...] = sv + b2r[...]

    out = pl.pallas_call(
        kern,
        grid=(g,),
        in_specs=[pl.BlockSpec((_BN, _HD), lambda i: (i, 0)),
                  pl.BlockSpec((_HD, 32), lambda i: (0, 0)),
                  pl.BlockSpec((1, 32), lambda i: (0, 0)),
                  pl.BlockSpec((1, 32), lambda i: (0, 0)),
                  pl.BlockSpec((1, 128), lambda i: (0, 0))],
        out_specs=pl.BlockSpec((_BN, 128), lambda i: (i, 0)),
        out_shape=jax.ShapeDtypeStruct((n, 128), jnp.float32),
    )(x, w1, b1.reshape(1, 32), w2t, b2)
    return out[:, 0]


def kernel(person_x, object_x, location_x, acts_edge, uses_src, uses_dst, at_src,
           at_dst, person_table, object_table, location_table, conv_Wn, conv_Wr,
           conv_b, cls_W1, cls_b1, cls_W2, cls_b2, sp_W1, sp_b1, sp_W2, sp_b2):
    f32 = jnp.float32
    xp = _embed(person_table, person_x, _NP)
    xo = _embed(object_table, object_x, _NO)
    xl = _embed(location_table, location_x, _NL)

    e0 = _pad_edges(acts_edge[0], acts_edge[1], _NP, _NP)
    e1 = _pad_edges(uses_src, uses_dst, _NP, _NO)
    e2 = _pad_edges(uses_dst, uses_src, _NO, _NP)
    e3 = _pad_edges(at_src, at_dst, _NP, _NL)
    e4 = _pad_edges(at_dst, at_src, _NL, _NP)

    c0, c1, c2, c3, c4 = _counts((e0, e1, e2, e3, e4))

    wr_p = (conv_Wr[:, 0] + conv_Wr[:, 2] + conv_Wr[:, 4]) / 3.0
    b_p = (conv_b[:, 0] + conv_b[:, 2] + conv_b[:, 4]) / 3.0

    for l in range(3):
        z0, z1, z3 = _mm_split(xp, [conv_Wn[l, 0], conv_Wn[l, 1], conv_Wn[l, 3]])
        (z2,) = _mm_split(xo, [conv_Wn[l, 2]])
        (z4,) = _mm_split(xl, [conv_Wn[l, 4]])
        s0 = _seg(z0.reshape(2 * _NP, 32), *e0)
        s1 = _seg(z1.reshape(2 * _NP, 32), *e1)
        s2 = _seg(z2.reshape(2 * _NO, 32), *e2)
        s3 = _seg(z3.reshape(2 * _NP, 32), *e3)
        s4 = _seg(z4.reshape(2 * _NL, 32), *e4)
        xp = _combine([s0, s2, s4], [c0, c2, c4], xp, wr_p[l], b_p[l], 1.0 / 3.0)
        xo = _combine([s1], [c1], xo, conv_Wr[l, 1], conv_b[l, 1], 1.0)
        xl = _combine([s3], [c3], xl, conv_Wr[l, 3], conv_b[l, 3], 1.0)

    sums = jnp.concatenate([_colsum(xp), _colsum(xo), _colsum(xl)], axis=1)
    scale = jnp.concatenate(
        [jnp.full((1, _HD), 1.0 / _NP, f32), jnp.full((1, _HD), 1.0 / _NO, f32),
         jnp.full((1, _HD), 1.0 / _NL, f32)], axis=1)
    ge, logits = _cls_head(sums, scale, cls_W1, cls_b1, cls_W2, cls_b2)

    b2full = jnp.full((1, 128), sp_b2[0], f32)
    scores = _scores(xp, sp_W1, sp_b1, sp_W2.reshape(1, 32), b2full)

    return logits, scores, ge.reshape(3 * _HD)


# speedup vs baseline: 9.3479x; 1.1345x over previous
import functools

import jax
import jax.numpy as jnp
from jax import lax
from jax.experimental import pallas as pl
from jax.experimental.pallas import tpu as pltpu
from jax.experimental.pallas import tpu_sc as plsc

_HD = 64
_NP, _NO, _NL = 50000, 20000, 10000
_E = 800000
_NC = 2
_NS = 16
_K = 4
_CH = _K * 128
_PT = 50176
_EPAD = _NS * _PT + _K * 128
_ROWS_PER_TILE = _PT // 128
_NMACRO = _ROWS_PER_TILE // _K


def _ndpad(n):
    return -(-(n + 16) // 256) * 256


@functools.lru_cache(None)
def _seg_kernel(n_src, nd_pad):
    zr = nd_pad // 16
    mesh = plsc.VectorSubcoreMesh(core_axis_name="c", subcore_axis_name="s")

    @functools.partial(
        pl.kernel,
        mesh=mesh,
        compiler_params=pltpu.CompilerParams(use_tc_tiling_on_sc=False),
        out_type=jax.ShapeDtypeStruct((2 * nd_pad, 32), jnp.float32),
        scratch_types=[
            pltpu.VMEM((_K, 128), jnp.int32),
            pltpu.VMEM((_K, 128), jnp.int32),
            pltpu.VMEM((_K, 128), jnp.int32),
            pltpu.VMEM((_K, 128), jnp.int32),
            pltpu.VMEM((_CH, 32), jnp.float32),
            pltpu.VMEM_SHARED((nd_pad, 32), jnp.float32),
            pltpu.SemaphoreType.DMA,
            pltpu.SemaphoreType.DMA,
        ],
    )
    def seg(z_hbm, src_hbm, dst_hbm, zero_hbm, out_hbm,
            src_a, dst_a, src_b, dst_b, rows, acc, sem, sem_idx):
        c = lax.axis_index("c")
        s = lax.axis_index("s")
        off = c * n_src
        r0 = s * zr
        pltpu.sync_copy(zero_hbm.at[pl.ds(r0, zr)], acc.at[pl.ds(r0, zr)])
        plsc.subcore_barrier()

        def fire_idx(m, srcv, dstv):
            rb = s * _ROWS_PER_TILE + m * _K
            pltpu.async_copy(src_hbm.at[pl.ds(rb, _K)], srcv, sem_idx)
            pltpu.async_copy(dst_hbm.at[pl.ds(rb, _K)], dstv, sem_idx)

        def wait_idx(srcv, dstv):
            pltpu.make_async_copy(src_hbm.at[pl.ds(0, _K)], srcv, sem_idx).wait()
            pltpu.make_async_copy(dst_hbm.at[pl.ds(0, _K)], dstv, sem_idx).wait()

        def adjust(srcv):
            for j in range(_K):
                for t in range(8):
                    srcv[j, pl.ds(t * 16, 16)] = srcv[j, pl.ds(t * 16, 16)] + off

        def fire(srcv, j):
            pltpu.async_copy(z_hbm.at[srcv.at[j]], rows.at[pl.ds(j * 128, 128)], sem)

        def drain(j):
            pltpu.make_async_copy(
                z_hbm.at[pl.ds(0, 128)], rows.at[pl.ds(j * 128, 128)], sem).wait()

        def chunk(m, cur_s, cur_d, nxt_s, nxt_d):
            wait_idx(nxt_s, nxt_d)
            adjust(nxt_s)
            for j in range(_K):
                drain(j)
            for j in range(_K):
                pltpu.sync_copy(rows.at[pl.ds(j * 128, 128)],
                                acc.at[cur_d.at[j]], add=True)
                fire(nxt_s, j)
            fire_idx(m + 2, cur_s, cur_d)

        pltpu.sync_copy(src_hbm.at[pl.ds(s * _ROWS_PER_TILE, _K)], src_a)
        pltpu.sync_copy(dst_hbm.at[pl.ds(s * _ROWS_PER_TILE, _K)], dst_a)
        adjust(src_a)
        for j in range(_K):
            fire(src_a, j)
        fire_idx(1, src_b, dst_b)

        def body(i, carry):
            chunk(2 * i, src_a, dst_a, src_b, dst_b)
            chunk(2 * i + 1, src_b, dst_b, src_a, dst_a)
            return carry

        lax.fori_loop(0, _NMACRO // 2, body, 0)
        wait_idx(src_b, dst_b)
        for j in range(_K):
            drain(j)
        plsc.subcore_barrier()
        pltpu.sync_copy(acc.at[pl.ds(r0, zr)], out_hbm.at[pl.ds(c * nd_pad + r0, zr)])

    return seg


def _seg(z2, src_rows, dst_rows, n_src, n_dst):
    nd_pad = _ndpad(n_dst)
    zeros = jnp.zeros((nd_pad, 32), jnp.float32)
    return _seg_kernel(n_src, nd_pad)(z2, src_rows, dst_rows, zeros)


_CNT_NDS = (_ndpad(_NP), _ndpad(_NO), _ndpad(_NP), _ndpad(_NL), _ndpad(_NP))


@functools.lru_cache(None)
def _counts_kernel():
    mesh = plsc.VectorSubcoreMesh(core_axis_name="c", subcore_axis_name="s")
    half = _ROWS_PER_TILE // 2
    nch = half // _K
    nd_max = max(_CNT_NDS)

    @functools.partial(
        pl.kernel,
        mesh=mesh,
        compiler_params=pltpu.CompilerParams(use_tc_tiling_on_sc=False),
        out_type=[jax.ShapeDtypeStruct((2 * nd, 16), jnp.float32)
                  for nd in _CNT_NDS],
        scratch_types=[
            pltpu.VMEM((_K, 128), jnp.int32),
            pltpu.VMEM((128, 16), jnp.float32),
            pltpu.VMEM_SHARED((nd_max, 16), jnp.float32),
        ],
    )
    def cnt(d0, d1, d2, d3, d4, zero_hbm, o0, o1, o2, o3, o4, dstv, ones, cacc):
        c = lax.axis_index("c")
        s = lax.axis_index("s")
        for i in range(128):
            ones[i, pl.ds(0, 16)] = jnp.full((16,), 1.0, jnp.float32)
        for dh, oh, nd in zip((d0, d1, d2, d3, d4), (o0, o1, o2, o3, o4),
                              _CNT_NDS):
            zrr = nd // 16
            r0 = s * zrr
            pltpu.sync_copy(zero_hbm.at[pl.ds(r0, zrr)], cacc.at[pl.ds(r0, zrr)])
            plsc.subcore_barrier()

            def body(m, carry, dh=dh):
                rb = s * _ROWS_PER_TILE + c * half + m * _K
                pltpu.sync_copy(dh.at[pl.ds(rb, _K)], dstv)
                for j in range(_K):
                    pltpu.sync_copy(ones, cacc.at[dstv.at[j]], add=True)
                return carry

            lax.fori_loop(0, nch, body, 0)
            plsc.subcore_barrier()
            pltpu.sync_copy(cacc.at[pl.ds(r0, zrr)],
                            oh.at[pl.ds(c * nd + r0, zrr)])
            plsc.subcore_barrier()

    return cnt


def _counts(edges):
    zeros = jnp.zeros((max(_CNT_NDS), 16), jnp.float32)
    outs = _counts_kernel()(*[e[1] for e in edges], zeros)
    return [o.reshape(2, nd, 16) for o, nd in zip(outs, _CNT_NDS)]


@functools.lru_cache(None)
def _embed_kernel(b_pad, wb, wk):
    mesh = plsc.VectorSubcoreMesh(core_axis_name="c", subcore_axis_name="s")

    @functools.partial(
        pl.kernel,
        mesh=mesh,
        compiler_params=pltpu.CompilerParams(use_tc_tiling_on_sc=False),
        out_type=jax.ShapeDtypeStruct((b_pad, _HD), jnp.float32),
        scratch_types=[
            pltpu.VMEM((wk, 128), jnp.int32),
            pltpu.VMEM((wb, _HD), jnp.float32),
            pltpu.SemaphoreType.DMA,
        ],
    )
    def emb(tab_hbm, idx_hbm, out_hbm, idxv, rows, sem):
        c = lax.axis_index("c")
        s = lax.axis_index("s")
        w = s * _NC + c
        pltpu.sync_copy(idx_hbm.at[w], idxv)
        cops = [
            pltpu.async_copy(tab_hbm.at[idxv.at[j]], rows.at[pl.ds(j * 128, 128)], sem)
            for j in range(wk)
        ]
        for cp in cops:
            cp.wait()
        pltpu.sync_copy(rows, out_hbm.at[pl.ds(w * wb, wb)])

    return emb


def _embed(table, idx, n):
    nw = _NC * _NS
    wb = -(-n // (nw * 128)) * 128
    b_pad = nw * wb
    wk = wb // 128
    pad = b_pad - n
    idx_p = jnp.concatenate(
        [idx.astype(jnp.int32), jnp.arange(pad, dtype=jnp.int32) % table.shape[0]]
    ).reshape(nw, wk, 128)
    out = _embed_kernel(b_pad, wb, wk)(table, idx_p)
    return out[:n]


def _pad_edges(src, dst, n_src, n_dst):
    pad = _EPAD - _E
    ar = jnp.arange(pad, dtype=jnp.int32)
    src_p = jnp.concatenate([src.astype(jnp.int32), ar % n_src])
    dst_p = jnp.concatenate([dst.astype(jnp.int32), n_dst + (ar % 16)])
    return (src_p.reshape(_EPAD // 128, 128), dst_p.reshape(_EPAD // 128, 128),
            n_src, n_dst)


_BN = 2000


def _mm_split(x, ws):
    n = x.shape[0]
    g = n // _BN
    nw = len(ws)

    def kern(*refs):
        x_ref = refs[0]
        w_refs = refs[1:1 + nw]
        o_refs = refs[1 + nw:]
        xb = x_ref[...]
        for wr, orf in zip(w_refs, o_refs):
            z = jnp.dot(xb, wr[...], preferred_element_type=jnp.float32)
            orf[0] = z[:, :32]
            orf[1] = z[:, 32:64]

    return pl.pallas_call(
        kern,
        grid=(g,),
        in_specs=[pl.BlockSpec((_BN, _HD), lambda i: (i, 0))]
        + [pl.BlockSpec((_HD, _HD), lambda i: (0, 0))] * nw,
        out_specs=[pl.BlockSpec((2, _BN, 32), lambda i: (0, i, 0))] * nw,
        out_shape=[jax.ShapeDtypeStruct((2, n, 32), jnp.float32)] * nw,
    )(x, *ws)


def _combine(accs, cnts, x, w, b, inv_nrel):
    n = x.shape[0]
    g = n // _BN
    nr = len(accs)

    def kern(*refs):
        a_refs = refs[:nr]
        c_refs = refs[nr:2 * nr]
        xr, wr, br = refs[2 * nr:2 * nr + 3]
        orf = refs[2 * nr + 3]
        m = None
        for ar, cr in zip(a_refs, c_refs):
            feat = jnp.concatenate([ar[0], ar[1]], axis=1)
            cnt = jnp.maximum(cr[0, :, :1] + cr[1, :, :1], 1.0)
            t = feat / cnt
            m = t if m is None else m + t
        pn = m * inv_nrel + jnp.dot(xr[...], wr[...],
                                    preferred_element_type=jnp.float32) + br[...]
        orf[...] = jnp.maximum(pn, 0.0) + xr[...]

    nd_pad = accs[0].shape[0] // 2
    a3 = [a.reshape(2, nd_pad, 32) for a in accs]
    c3 = list(cnts)
    return pl.pallas_call(
        kern,
        grid=(g,),
        in_specs=(
            [pl.BlockSpec((2, _BN, 32), lambda i: (0, i, 0))] * nr
            + [pl.BlockSpec((2, _BN, 16), lambda i: (0, i, 0))] * nr
            + [pl.BlockSpec((_BN, _HD), lambda i: (i, 0)),
               pl.BlockSpec((_HD, _HD), lambda i: (0, 0)),
               pl.BlockSpec((1, _HD), lambda i: (0, 0))]
        ),
        out_specs=pl.BlockSpec((_BN, _HD), lambda i: (i, 0)),
        out_shape=jax.ShapeDtypeStruct((n, _HD), jnp.float32),
    )(*a3, *c3, x, w, b.reshape(1, _HD))


def _colsum(x):
    n = x.shape[0]
    g = n // _BN

    def kern(xr, orf):
        @pl.when(pl.program_id(0) == 0)
        def _():
            orf[...] = jnp.zeros_like(orf)

        orf[...] += jnp.sum(xr[...], axis=0, keepdims=True)

    return pl.pallas_call(
        kern,
        grid=(g,),
        in_specs=[pl.BlockSpec((_BN, _HD), lambda i: (i, 0))],
        out_specs=pl.BlockSpec((1, _HD), lambda i: (0, 0)),
        out_shape=jax.ShapeDtypeStruct((1, _HD), jnp.float32),
    )(x)


def _cls_head(sums, scale, w1, b1, w2, b2):
    def kern(sr, scr, w1r, b1r, w2r, b2r, oge, olg):
        ge = sr[...] * scr[...]
        oge[...] = ge
        h = jnp.maximum(
            jnp.dot(ge, w1r[...], preferred_element_type=jnp.float32) + b1r[...], 0.0)
        olg[...] = jnp.dot(h, w2r[...], preferred_element_type=jnp.float32) + b2r[...]

    return pl.pallas_call(
        kern,
        out_shape=[jax.ShapeDtypeStruct((1, 3 * _HD), jnp.float32),
                   jax.ShapeDtypeStruct((1, 50), jnp.float32)],
    )(sums, scale, w1, b1.reshape(1, _HD), w2, b2.reshape(1, 50))


def _scores(x, w1, b1, w2t, b2):
    n = x.shape[0]
    g = n // _BN

    def kern(xr, w1r, b1r, w2r, b2r, orf):
        h = jnp.maximum(
            jnp.dot(xr[...], w1r[...], preferred_element_type=jnp.float32) + b1r[...],
            0.0)
        sv = jnp.sum(h * w2r[...], axis=1, keepdims=True)
        orf[...] = sv + b2r[...]

    out = pl.pallas_call(
        kern,
        grid=(g,),
        in_specs=[pl.BlockSpec((_BN, _HD), lambda i: (i, 0)),
                  pl.BlockSpec((_HD, 32), lambda i: (0, 0)),
                  pl.BlockSpec((1, 32), lambda i: (0, 0)),
                  pl.BlockSpec((1, 32), lambda i: (0, 0)),
                  pl.BlockSpec((1, 128), lambda i: (0, 0))],
        out_specs=pl.BlockSpec((_BN, 128), lambda i: (i, 0)),
        out_shape=jax.ShapeDtypeStruct((n, 128), jnp.float32),
    )(x, w1, b1.reshape(1, 32), w2t, b2)
    return out[:, 0]


def kernel(person_x, object_x, location_x, acts_edge, uses_src, uses_dst, at_src,
           at_dst, person_table, object_table, location_table, conv_Wn, conv_Wr,
           conv_b, cls_W1, cls_b1, cls_W2, cls_b2, sp_W1, sp_b1, sp_W2, sp_b2):
    f32 = jnp.float32
    xp = _embed(person_table, person_x, _NP)
    xo = _embed(object_table, object_x, _NO)
    xl = _embed(location_table, location_x, _NL)

    e0 = _pad_edges(acts_edge[0], acts_edge[1], _NP, _NP)
    e1 = _pad_edges(uses_src, uses_dst, _NP, _NO)
    e2 = _pad_edges(uses_dst, uses_src, _NO, _NP)
    e3 = _pad_edges(at_src, at_dst, _NP, _NL)
    e4 = _pad_edges(at_dst, at_src, _NL, _NP)

    c0, c1, c2, c3, c4 = _counts((e0, e1, e2, e3, e4))

    wr_p = (conv_Wr[:, 0] + conv_Wr[:, 2] + conv_Wr[:, 4]) / 3.0
    b_p = (conv_b[:, 0] + conv_b[:, 2] + conv_b[:, 4]) / 3.0

    for l in range(3):
        z0, z1, z3 = _mm_split(xp, [conv_Wn[l, 0], conv_Wn[l, 1], conv_Wn[l, 3]])
        (z2,) = _mm_split(xo, [conv_Wn[l, 2]])
        (z4,) = _mm_split(xl, [conv_Wn[l, 4]])
        s0 = _seg(z0.reshape(2 * _NP, 32), *e0)
        s1 = _seg(z1.reshape(2 * _NP, 32), *e1)
        s2 = _seg(z2.reshape(2 * _NO, 32), *e2)
        s3 = _seg(z3.reshape(2 * _NP, 32), *e3)
        s4 = _seg(z4.reshape(2 * _NL, 32), *e4)
        xp = _combine([s0, s2, s4], [c0, c2, c4], xp, wr_p[l], b_p[l], 1.0 / 3.0)
        xo = _combine([s1], [c1], xo, conv_Wr[l, 1], conv_b[l, 1], 1.0)
        xl = _combine([s3], [c3], xl, conv_Wr[l, 3], conv_b[l, 3], 1.0)

    sums = jnp.concatenate([_colsum(xp), _colsum(xo), _colsum(xl)], axis=1)
    scale = jnp.concatenate(
        [jnp.full((1, _HD), 1.0 / _NP, f32), jnp.full((1, _HD), 1.0 / _NO, f32),
         jnp.full((1, _HD), 1.0 / _NL, f32)], axis=1)
    ge, logits = _cls_head(sums, scale, cls_W1, cls_b1, cls_W2, cls_b2)

    b2full = jnp.full((1, 128), sp_b2[0], f32)
    scores = _scores(xp, sp_W1, sp_b1, sp_W2.reshape(1, 32), b2full)

    return logits, scores, ge.reshape(3 * _HD)


# speedup vs baseline: 9.8255x; 1.0511x over previous
import functools

import jax
import jax.numpy as jnp
from jax import lax
from jax.experimental import pallas as pl
from jax.experimental.pallas import tpu as pltpu
from jax.experimental.pallas import tpu_sc as plsc

_HD = 64
_NP, _NO, _NL = 50000, 20000, 10000
_E = 800000
_NC = 2
_NS = 16
_K = 4
_CH = _K * 128
_PT = 50176
_EPAD = _NS * _PT + _K * 128
_ROWS_PER_TILE = _PT // 128
_NMACRO = _ROWS_PER_TILE // _K


def _ndpad(n):
    return -(-(n + 16) // 256) * 256


@functools.lru_cache(None)
def _seg_kernel(n_src, nd_pad):
    zr = nd_pad // 16
    mesh = plsc.VectorSubcoreMesh(core_axis_name="c", subcore_axis_name="s")

    @functools.partial(
        pl.kernel,
        mesh=mesh,
        compiler_params=pltpu.CompilerParams(use_tc_tiling_on_sc=False),
        out_type=jax.ShapeDtypeStruct((2 * nd_pad, 32), jnp.float32),
        scratch_types=[
            pltpu.VMEM((_K, 128), jnp.int32),
            pltpu.VMEM((_K, 128), jnp.int32),
            pltpu.VMEM((_K, 128), jnp.int32),
            pltpu.VMEM((_K, 128), jnp.int32),
            pltpu.VMEM((_CH, 32), jnp.float32),
            pltpu.VMEM_SHARED((nd_pad, 32), jnp.float32),
            pltpu.SemaphoreType.DMA,
            pltpu.SemaphoreType.DMA,
            pltpu.SemaphoreType.DMA,
        ],
    )
    def seg(z_hbm, src_hbm, dst_hbm, zero_hbm, out_hbm,
            src_a, dst_a, src_b, dst_b, rows, acc, sem, sem_idx, sem_s):
        c = lax.axis_index("c")
        s = lax.axis_index("s")
        off = c * n_src
        r0 = s * zr
        pltpu.sync_copy(zero_hbm.at[pl.ds(r0, zr)], acc.at[pl.ds(r0, zr)])
        plsc.subcore_barrier()

        def fire_idx(m, srcv, dstv):
            rb = s * _ROWS_PER_TILE + m * _K
            pltpu.async_copy(src_hbm.at[pl.ds(rb, _K)], srcv, sem_idx)
            pltpu.async_copy(dst_hbm.at[pl.ds(rb, _K)], dstv, sem_idx)

        def wait_idx(srcv, dstv):
            pltpu.make_async_copy(src_hbm.at[pl.ds(0, _K)], srcv, sem_idx).wait()
            pltpu.make_async_copy(dst_hbm.at[pl.ds(0, _K)], dstv, sem_idx).wait()

        def adjust(srcv):
            for j in range(_K):
                for t in range(8):
                    srcv[j, pl.ds(t * 16, 16)] = srcv[j, pl.ds(t * 16, 16)] + off

        def fire(srcv, j):
            pltpu.async_copy(z_hbm.at[srcv.at[j]], rows.at[pl.ds(j * 128, 128)], sem)

        def drain(j):
            pltpu.make_async_copy(
                z_hbm.at[pl.ds(0, 128)], rows.at[pl.ds(j * 128, 128)], sem).wait()

        def chunk(m, cur_s, cur_d, nxt_s, nxt_d):
            wait_idx(nxt_s, nxt_d)
            adjust(nxt_s)
            for j in range(_K):
                drain(j)
            for j in range(_K):
                pltpu.async_copy(rows.at[pl.ds(j * 128, 128)],
                                 acc.at[cur_d.at[j]], sem_s, add=True)
            for j in range(_K):
                pltpu.make_async_copy(rows.at[pl.ds(j * 128, 128)],
                                      acc.at[pl.ds(0, 128)], sem_s).wait()
                fire(nxt_s, j)
            fire_idx(m + 2, cur_s, cur_d)

        pltpu.sync_copy(src_hbm.at[pl.ds(s * _ROWS_PER_TILE, _K)], src_a)
        pltpu.sync_copy(dst_hbm.at[pl.ds(s * _ROWS_PER_TILE, _K)], dst_a)
        adjust(src_a)
        for j in range(_K):
            fire(src_a, j)
        fire_idx(1, src_b, dst_b)

        def body(i, carry):
            chunk(2 * i, src_a, dst_a, src_b, dst_b)
            chunk(2 * i + 1, src_b, dst_b, src_a, dst_a)
            return carry

        lax.fori_loop(0, _NMACRO // 2, body, 0)
        wait_idx(src_b, dst_b)
        for j in range(_K):
            drain(j)
        plsc.subcore_barrier()
        pltpu.sync_copy(acc.at[pl.ds(r0, zr)], out_hbm.at[pl.ds(c * nd_pad + r0, zr)])

    return seg


def _seg(z2, src_rows, dst_rows, n_src, n_dst):
    nd_pad = _ndpad(n_dst)
    zeros = jnp.zeros((nd_pad, 32), jnp.float32)
    return _seg_kernel(n_src, nd_pad)(z2, src_rows, dst_rows, zeros)


_CNT_NDS = (_ndpad(_NP), _ndpad(_NO), _ndpad(_NP), _ndpad(_NL), _ndpad(_NP))


@functools.lru_cache(None)
def _counts_kernel():
    mesh = plsc.VectorSubcoreMesh(core_axis_name="c", subcore_axis_name="s")
    half = _ROWS_PER_TILE // 2
    nch = half // _K
    nd_max = max(_CNT_NDS)

    @functools.partial(
        pl.kernel,
        mesh=mesh,
        compiler_params=pltpu.CompilerParams(use_tc_tiling_on_sc=False),
        out_type=[jax.ShapeDtypeStruct((2 * nd, 16), jnp.float32)
                  for nd in _CNT_NDS],
        scratch_types=[
            pltpu.VMEM((_K, 128), jnp.int32),
            pltpu.VMEM((128, 16), jnp.float32),
            pltpu.VMEM_SHARED((nd_max, 16), jnp.float32),
            pltpu.SemaphoreType.DMA,
        ],
    )
    def cnt(d0, d1, d2, d3, d4, zero_hbm, o0, o1, o2, o3, o4, dstv, ones, cacc,
            sem_s):
        c = lax.axis_index("c")
        s = lax.axis_index("s")
        for i in range(128):
            ones[i, pl.ds(0, 16)] = jnp.full((16,), 1.0, jnp.float32)
        for dh, oh, nd in zip((d0, d1, d2, d3, d4), (o0, o1, o2, o3, o4),
                              _CNT_NDS):
            zrr = nd // 16
            r0 = s * zrr
            pltpu.sync_copy(zero_hbm.at[pl.ds(r0, zrr)], cacc.at[pl.ds(r0, zrr)])
            plsc.subcore_barrier()

            def body(m, carry, dh=dh):
                rb = s * _ROWS_PER_TILE + c * half + m * _K
                pltpu.sync_copy(dh.at[pl.ds(rb, _K)], dstv)
                for j in range(_K):
                    pltpu.async_copy(ones, cacc.at[dstv.at[j]], sem_s, add=True)
                for j in range(_K):
                    pltpu.make_async_copy(ones, cacc.at[pl.ds(0, 128)],
                                          sem_s).wait()
                return carry

            lax.fori_loop(0, nch, body, 0)
            plsc.subcore_barrier()
            pltpu.sync_copy(cacc.at[pl.ds(r0, zrr)],
                            oh.at[pl.ds(c * nd + r0, zrr)])
            plsc.subcore_barrier()

    return cnt


def _counts(edges):
    zeros = jnp.zeros((max(_CNT_NDS), 16), jnp.float32)
    outs = _counts_kernel()(*[e[1] for e in edges], zeros)
    return [o.reshape(2, nd, 16) for o, nd in zip(outs, _CNT_NDS)]


@functools.lru_cache(None)
def _embed_kernel(b_pad, wb, wk):
    mesh = plsc.VectorSubcoreMesh(core_axis_name="c", subcore_axis_name="s")

    @functools.partial(
        pl.kernel,
        mesh=mesh,
        compiler_params=pltpu.CompilerParams(use_tc_tiling_on_sc=False),
        out_type=jax.ShapeDtypeStruct((b_pad, _HD), jnp.float32),
        scratch_types=[
            pltpu.VMEM((wk, 128), jnp.int32),
            pltpu.VMEM((wb, _HD), jnp.float32),
            pltpu.SemaphoreType.DMA,
        ],
    )
    def emb(tab_hbm, idx_hbm, out_hbm, idxv, rows, sem):
        c = lax.axis_index("c")
        s = lax.axis_index("s")
        w = s * _NC + c
        pltpu.sync_copy(idx_hbm.at[w], idxv)
        cops = [
            pltpu.async_copy(tab_hbm.at[idxv.at[j]], rows.at[pl.ds(j * 128, 128)], sem)
            for j in range(wk)
        ]
        for cp in cops:
            cp.wait()
        pltpu.sync_copy(rows, out_hbm.at[pl.ds(w * wb, wb)])

    return emb


def _embed(table, idx, n):
    nw = _NC * _NS
    wb = -(-n // (nw * 128)) * 128
    b_pad = nw * wb
    wk = wb // 128
    pad = b_pad - n
    idx_p = jnp.concatenate(
        [idx.astype(jnp.int32), jnp.arange(pad, dtype=jnp.int32) % table.shape[0]]
    ).reshape(nw, wk, 128)
    out = _embed_kernel(b_pad, wb, wk)(table, idx_p)
    return out[:n]


def _pad_edges(src, dst, n_src, n_dst):
    pad = _EPAD - _E
    ar = jnp.arange(pad, dtype=jnp.int32)
    src_p = jnp.concatenate([src.astype(jnp.int32), ar % n_src])
    dst_p = jnp.concatenate([dst.astype(jnp.int32), n_dst + (ar % 16)])
    return (src_p.reshape(_EPAD // 128, 128), dst_p.reshape(_EPAD // 128, 128),
            n_src, n_dst)


_BN = 2000


def _mm_split(x, ws):
    n = x.shape[0]
    g = n // _BN
    nw = len(ws)

    def kern(*refs):
        x_ref = refs[0]
        w_refs = refs[1:1 + nw]
        o_refs = refs[1 + nw:]
        xb = x_ref[...]
        for wr, orf in zip(w_refs, o_refs):
            z = jnp.dot(xb, wr[...], preferred_element_type=jnp.float32)
            orf[0] = z[:, :32]
            orf[1] = z[:, 32:64]

    return pl.pallas_call(
        kern,
        grid=(g,),
        in_specs=[pl.BlockSpec((_BN, _HD), lambda i: (i, 0))]
        + [pl.BlockSpec((_HD, _HD), lambda i: (0, 0))] * nw,
        out_specs=[pl.BlockSpec((2, _BN, 32), lambda i: (0, i, 0))] * nw,
        out_shape=[jax.ShapeDtypeStruct((2, n, 32), jnp.float32)] * nw,
    )(x, *ws)


def _combine(accs, cnts, x, w, b, inv_nrel):
    n = x.shape[0]
    g = n // _BN
    nr = len(accs)

    def kern(*refs):
        a_refs = refs[:nr]
        c_refs = refs[nr:2 * nr]
        xr, wr, br = refs[2 * nr:2 * nr + 3]
        orf = refs[2 * nr + 3]
        m = None
        for ar, cr in zip(a_refs, c_refs):
            feat = jnp.concatenate([ar[0], ar[1]], axis=1)
            cnt = jnp.maximum(cr[0, :, :1] + cr[1, :, :1], 1.0)
            t = feat / cnt
            m = t if m is None else m + t
        pn = m * inv_nrel + jnp.dot(xr[...], wr[...],
                                    preferred_element_type=jnp.float32) + br[...]
        orf[...] = jnp.maximum(pn, 0.0) + xr[...]

    nd_pad = accs[0].shape[0] // 2
    a3 = [a.reshape(2, nd_pad, 32) for a in accs]
    c3 = list(cnts)
    return pl.pallas_call(
        kern,
        grid=(g,),
        in_specs=(
            [pl.BlockSpec((2, _BN, 32), lambda i: (0, i, 0))] * nr
            + [pl.BlockSpec((2, _BN, 16), lambda i: (0, i, 0))] * nr
            + [pl.BlockSpec((_BN, _HD), lambda i: (i, 0)),
               pl.BlockSpec((_HD, _HD), lambda i: (0, 0)),
               pl.BlockSpec((1, _HD), lambda i: (0, 0))]
        ),
        out_specs=pl.BlockSpec((_BN, _HD), lambda i: (i, 0)),
        out_shape=jax.ShapeDtypeStruct((n, _HD), jnp.float32),
    )(*a3, *c3, x, w, b.reshape(1, _HD))


def _colsum(x):
    n = x.shape[0]
    g = n // _BN

    def kern(xr, orf):
        @pl.when(pl.program_id(0) == 0)
        def _():
            orf[...] = jnp.zeros_like(orf)

        orf[...] += jnp.sum(xr[...], axis=0, keepdims=True)

    return pl.pallas_call(
        kern,
        grid=(g,),
        in_specs=[pl.BlockSpec((_BN, _HD), lambda i: (i, 0))],
        out_specs=pl.BlockSpec((1, _HD), lambda i: (0, 0)),
        out_shape=jax.ShapeDtypeStruct((1, _HD), jnp.float32),
    )(x)


def _cls_head(sums, scale, w1, b1, w2, b2):
    def kern(sr, scr, w1r, b1r, w2r, b2r, oge, olg):
        ge = sr[...] * scr[...]
        oge[...] = ge
        h = jnp.maximum(
            jnp.dot(ge, w1r[...], preferred_element_type=jnp.float32) + b1r[...], 0.0)
        olg[...] = jnp.dot(h, w2r[...], preferred_element_type=jnp.float32) + b2r[...]

    return pl.pallas_call(
        kern,
        out_shape=[jax.ShapeDtypeStruct((1, 3 * _HD), jnp.float32),
                   jax.ShapeDtypeStruct((1, 50), jnp.float32)],
    )(sums, scale, w1, b1.reshape(1, _HD), w2, b2.reshape(1, 50))


def _scores(x, w1, b1, w2t, b2):
    n = x.shape[0]
    g = n // _BN

    def kern(xr, w1r, b1r, w2r, b2r, orf):
        h = jnp.maximum(
            jnp.dot(xr[...], w1r[...], preferred_element_type=jnp.float32) + b1r[...],
            0.0)
        sv = jnp.sum(h * w2r[...], axis=1, keepdims=True)
        orf[...] = sv + b2r[...]

    out = pl.pallas_call(
        kern,
        grid=(g,),
        in_specs=[pl.BlockSpec((_BN, _HD), lambda i: (i, 0)),
                  pl.BlockSpec((_HD, 32), lambda i: (0, 0)),
                  pl.BlockSpec((1, 32), lambda i: (0, 0)),
                  pl.BlockSpec((1, 32), lambda i: (0, 0)),
                  pl.BlockSpec((1, 128), lambda i: (0, 0))],
        out_specs=pl.BlockSpec((_BN, 128), lambda i: (i, 0)),
        out_shape=jax.ShapeDtypeStruct((n, 128), jnp.float32),
    )(x, w1, b1.reshape(1, 32), w2t, b2)
    return out[:, 0]


def kernel(person_x, object_x, location_x, acts_edge, uses_src, uses_dst, at_src,
           at_dst, person_table, object_table, location_table, conv_Wn, conv_Wr,
           conv_b, cls_W1, cls_b1, cls_W2, cls_b2, sp_W1, sp_b1, sp_W2, sp_b2):
    f32 = jnp.float32
    xp = _embed(person_table, person_x, _NP)
    xo = _embed(object_table, object_x, _NO)
    xl = _embed(location_table, location_x, _NL)

    e0 = _pad_edges(acts_edge[0], acts_edge[1], _NP, _NP)
    e1 = _pad_edges(uses_src, uses_dst, _NP, _NO)
    e2 = _pad_edges(uses_dst, uses_src, _NO, _NP)
    e3 = _pad_edges(at_src, at_dst, _NP, _NL)
    e4 = _pad_edges(at_dst, at_src, _NL, _NP)

    c0, c1, c2, c3, c4 = _counts((e0, e1, e2, e3, e4))

    wr_p = (conv_Wr[:, 0] + conv_Wr[:, 2] + conv_Wr[:, 4]) / 3.0
    b_p = (conv_b[:, 0] + conv_b[:, 2] + conv_b[:, 4]) / 3.0

    for l in range(3):
        z0, z1, z3 = _mm_split(xp, [conv_Wn[l, 0], conv_Wn[l, 1], conv_Wn[l, 3]])
        (z2,) = _mm_split(xo, [conv_Wn[l, 2]])
        (z4,) = _mm_split(xl, [conv_Wn[l, 4]])
        s0 = _seg(z0.reshape(2 * _NP, 32), *e0)
        s1 = _seg(z1.reshape(2 * _NP, 32), *e1)
        s2 = _seg(z2.reshape(2 * _NO, 32), *e2)
        s3 = _seg(z3.reshape(2 * _NP, 32), *e3)
        s4 = _seg(z4.reshape(2 * _NL, 32), *e4)
        xp = _combine([s0, s2, s4], [c0, c2, c4], xp, wr_p[l], b_p[l], 1.0 / 3.0)
        xo = _combine([s1], [c1], xo, conv_Wr[l, 1], conv_b[l, 1], 1.0)
        xl = _combine([s3], [c3], xl, conv_Wr[l, 3], conv_b[l, 3], 1.0)

    sums = jnp.concatenate([_colsum(xp), _colsum(xo), _colsum(xl)], axis=1)
    scale = jnp.concatenate(
        [jnp.full((1, _HD), 1.0 / _NP, f32), jnp.full((1, _HD), 1.0 / _NO, f32),
         jnp.full((1, _HD), 1.0 / _NL, f32)], axis=1)
    ge, logits = _cls_head(sums, scale, cls_W1, cls_b1, cls_W2, cls_b2)

    b2full = jnp.full((1, 128), sp_b2[0], f32)
    scores = _scores(xp, sp_W1, sp_b1, sp_W2.reshape(1, 32), b2full)

    return logits, scores, ge.reshape(3 * _HD)


# speedup vs baseline: 10.2470x; 1.0429x over previous
import functools

import jax
import jax.numpy as jnp
from jax import lax
from jax.experimental import pallas as pl
from jax.experimental.pallas import tpu as pltpu
from jax.experimental.pallas import tpu_sc as plsc

_HD = 64
_NP, _NO, _NL = 50000, 20000, 10000
_E = 800000
_NC = 2
_NS = 16
_K = 4
_KBIG = 14
_PT = 50176
_EPAD = _NS * _PT + 2 * _KBIG * 128
_ROWS_PER_TILE = _PT // 128
_NMACRO = _ROWS_PER_TILE // _K


def _ndpad(n):
    return -(-(n + 16) // 256) * 256


@functools.lru_cache(None)
def _seg_kernel(n_src, nd_pad):
    zr = nd_pad // 16
    k = _K if nd_pad > 30000 else _KBIG
    nmacro = _ROWS_PER_TILE // k
    mesh = plsc.VectorSubcoreMesh(core_axis_name="c", subcore_axis_name="s")

    @functools.partial(
        pl.kernel,
        mesh=mesh,
        compiler_params=pltpu.CompilerParams(use_tc_tiling_on_sc=False),
        out_type=jax.ShapeDtypeStruct((2 * nd_pad, 32), jnp.float32),
        scratch_types=[
            pltpu.VMEM((k, 128), jnp.int32),
            pltpu.VMEM((k, 128), jnp.int32),
            pltpu.VMEM((k, 128), jnp.int32),
            pltpu.VMEM((k, 128), jnp.int32),
            pltpu.VMEM((k * 128, 32), jnp.float32),
            pltpu.VMEM_SHARED((nd_pad, 32), jnp.float32),
            pltpu.SemaphoreType.DMA,
            pltpu.SemaphoreType.DMA,
            pltpu.SemaphoreType.DMA,
        ],
    )
    def seg(z_hbm, src_hbm, dst_hbm, zero_hbm, out_hbm,
            src_a, dst_a, src_b, dst_b, rows, acc, sem, sem_idx, sem_s):
        c = lax.axis_index("c")
        s = lax.axis_index("s")
        off = c * n_src
        r0 = s * zr
        pltpu.sync_copy(zero_hbm.at[pl.ds(r0, zr)], acc.at[pl.ds(r0, zr)])
        plsc.subcore_barrier()

        def fire_idx(m, srcv, dstv):
            rb = s * _ROWS_PER_TILE + m * k
            pltpu.async_copy(src_hbm.at[pl.ds(rb, k)], srcv, sem_idx)
            pltpu.async_copy(dst_hbm.at[pl.ds(rb, k)], dstv, sem_idx)

        def wait_idx(srcv, dstv):
            pltpu.make_async_copy(src_hbm.at[pl.ds(0, k)], srcv, sem_idx).wait()
            pltpu.make_async_copy(dst_hbm.at[pl.ds(0, k)], dstv, sem_idx).wait()

        def adjust(srcv):
            for j in range(k):
                for t in range(8):
                    srcv[j, pl.ds(t * 16, 16)] = srcv[j, pl.ds(t * 16, 16)] + off

        def fire(srcv, j):
            pltpu.async_copy(z_hbm.at[srcv.at[j]], rows.at[pl.ds(j * 128, 128)], sem)

        def drain(j):
            pltpu.make_async_copy(
                z_hbm.at[pl.ds(0, 128)], rows.at[pl.ds(j * 128, 128)], sem).wait()

        def chunk(m, cur_s, cur_d, nxt_s, nxt_d):
            wait_idx(nxt_s, nxt_d)
            adjust(nxt_s)
            for j in range(k):
                drain(j)
            for j in range(k):
                pltpu.async_copy(rows.at[pl.ds(j * 128, 128)],
                                 acc.at[cur_d.at[j]], sem_s, add=True)
            for j in range(k):
                pltpu.make_async_copy(rows.at[pl.ds(j * 128, 128)],
                                      acc.at[pl.ds(0, 128)], sem_s).wait()
                fire(nxt_s, j)
            fire_idx(m + 2, cur_s, cur_d)

        pltpu.sync_copy(src_hbm.at[pl.ds(s * _ROWS_PER_TILE, k)], src_a)
        pltpu.sync_copy(dst_hbm.at[pl.ds(s * _ROWS_PER_TILE, k)], dst_a)
        adjust(src_a)
        for j in range(k):
            fire(src_a, j)
        fire_idx(1, src_b, dst_b)

        def body(i, carry):
            chunk(2 * i, src_a, dst_a, src_b, dst_b)
            chunk(2 * i + 1, src_b, dst_b, src_a, dst_a)
            return carry

        lax.fori_loop(0, nmacro // 2, body, 0)
        wait_idx(src_b, dst_b)
        for j in range(k):
            drain(j)
        plsc.subcore_barrier()
        pltpu.sync_copy(acc.at[pl.ds(r0, zr)], out_hbm.at[pl.ds(c * nd_pad + r0, zr)])

    return seg


def _seg(z2, src_rows, dst_rows, n_src, n_dst):
    nd_pad = _ndpad(n_dst)
    zeros = jnp.zeros((nd_pad, 32), jnp.float32)
    return _seg_kernel(n_src, nd_pad)(z2, src_rows, dst_rows, zeros)


_CNT_NDS = (_ndpad(_NP), _ndpad(_NO), _ndpad(_NP), _ndpad(_NL), _ndpad(_NP))


@functools.lru_cache(None)
def _counts_kernel():
    mesh = plsc.VectorSubcoreMesh(core_axis_name="c", subcore_axis_name="s")
    half = _ROWS_PER_TILE // 2
    nch = half // _K
    nd_max = max(_CNT_NDS)

    @functools.partial(
        pl.kernel,
        mesh=mesh,
        compiler_params=pltpu.CompilerParams(use_tc_tiling_on_sc=False),
        out_type=[jax.ShapeDtypeStruct((2 * nd, 16), jnp.float32)
                  for nd in _CNT_NDS],
        scratch_types=[
            pltpu.VMEM((_K, 128), jnp.int32),
            pltpu.VMEM((128, 16), jnp.float32),
            pltpu.VMEM_SHARED((nd_max, 16), jnp.float32),
            pltpu.SemaphoreType.DMA,
        ],
    )
    def cnt(d0, d1, d2, d3, d4, zero_hbm, o0, o1, o2, o3, o4, dstv, ones, cacc,
            sem_s):
        c = lax.axis_index("c")
        s = lax.axis_index("s")
        for i in range(128):
            ones[i, pl.ds(0, 16)] = jnp.full((16,), 1.0, jnp.float32)
        for dh, oh, nd in zip((d0, d1, d2, d3, d4), (o0, o1, o2, o3, o4),
                              _CNT_NDS):
            zrr = nd // 16
            r0 = s * zrr
            pltpu.sync_copy(zero_hbm.at[pl.ds(r0, zrr)], cacc.at[pl.ds(r0, zrr)])
            plsc.subcore_barrier()

            def body(m, carry, dh=dh):
                rb = s * _ROWS_PER_TILE + c * half + m * _K
                pltpu.sync_copy(dh.at[pl.ds(rb, _K)], dstv)
                for j in range(_K):
                    pltpu.async_copy(ones, cacc.at[dstv.at[j]], sem_s, add=True)
                for j in range(_K):
                    pltpu.make_async_copy(ones, cacc.at[pl.ds(0, 128)],
                                          sem_s).wait()
                return carry

            lax.fori_loop(0, nch, body, 0)
            plsc.subcore_barrier()
            pltpu.sync_copy(cacc.at[pl.ds(r0, zrr)],
                            oh.at[pl.ds(c * nd + r0, zrr)])
            plsc.subcore_barrier()

    return cnt


def _counts(edges):
    zeros = jnp.zeros((max(_CNT_NDS), 16), jnp.float32)
    outs = _counts_kernel()(*[e[1] for e in edges], zeros)
    return [o.reshape(2, nd, 16) for o, nd in zip(outs, _CNT_NDS)]


@functools.lru_cache(None)
def _embed_kernel(b_pad, wb, wk):
    mesh = plsc.VectorSubcoreMesh(core_axis_name="c", subcore_axis_name="s")

    @functools.partial(
        pl.kernel,
        mesh=mesh,
        compiler_params=pltpu.CompilerParams(use_tc_tiling_on_sc=False),
        out_type=jax.ShapeDtypeStruct((b_pad, _HD), jnp.float32),
        scratch_types=[
            pltpu.VMEM((wk, 128), jnp.int32),
            pltpu.VMEM((wb, _HD), jnp.float32),
            pltpu.SemaphoreType.DMA,
        ],
    )
    def emb(tab_hbm, idx_hbm, out_hbm, idxv, rows, sem):
        c = lax.axis_index("c")
        s = lax.axis_index("s")
        w = s * _NC + c
        pltpu.sync_copy(idx_hbm.at[w], idxv)
        cops = [
            pltpu.async_copy(tab_hbm.at[idxv.at[j]], rows.at[pl.ds(j * 128, 128)], sem)
            for j in range(wk)
        ]
        for cp in cops:
            cp.wait()
        pltpu.sync_copy(rows, out_hbm.at[pl.ds(w * wb, wb)])

    return emb


def _embed(table, idx, n):
    nw = _NC * _NS
    wb = -(-n // (nw * 128)) * 128
    b_pad = nw * wb
    wk = wb // 128
    pad = b_pad - n
    idx_p = jnp.concatenate(
        [idx.astype(jnp.int32), jnp.arange(pad, dtype=jnp.int32) % table.shape[0]]
    ).reshape(nw, wk, 128)
    out = _embed_kernel(b_pad, wb, wk)(table, idx_p)
    return out[:n]


def _pad_edges(src, dst, n_src, n_dst):
    pad = _EPAD - _E
    ar = jnp.arange(pad, dtype=jnp.int32)
    src_p = jnp.concatenate([src.astype(jnp.int32), ar % n_src])
    dst_p = jnp.concatenate([dst.astype(jnp.int32), n_dst + (ar % 16)])
    return (src_p.reshape(_EPAD // 128, 128), dst_p.reshape(_EPAD // 128, 128),
            n_src, n_dst)


_BN = 2000


def _mm_split(x, ws):
    n = x.shape[0]
    g = n // _BN
    nw = len(ws)

    def kern(*refs):
        x_ref = refs[0]
        w_refs = refs[1:1 + nw]
        o_refs = refs[1 + nw:]
        xb = x_ref[...]
        for wr, orf in zip(w_refs, o_refs):
            z = jnp.dot(xb, wr[...], preferred_element_type=jnp.float32)
            orf[0] = z[:, :32]
            orf[1] = z[:, 32:64]

    return pl.pallas_call(
        kern,
        grid=(g,),
        in_specs=[pl.BlockSpec((_BN, _HD), lambda i: (i, 0))]
        + [pl.BlockSpec((_HD, _HD), lambda i: (0, 0))] * nw,
        out_specs=[pl.BlockSpec((2, _BN, 32), lambda i: (0, i, 0))] * nw,
        out_shape=[jax.ShapeDtypeStruct((2, n, 32), jnp.float32)] * nw,
    )(x, *ws)


def _combine(accs, cnts, x, w, b, inv_nrel):
    n = x.shape[0]
    g = n // _BN
    nr = len(accs)

    def kern(*refs):
        a_refs = refs[:nr]
        c_refs = refs[nr:2 * nr]
        xr, wr, br = refs[2 * nr:2 * nr + 3]
        orf = refs[2 * nr + 3]
        m = None
        for ar, cr in zip(a_refs, c_refs):
            feat = jnp.concatenate([ar[0], ar[1]], axis=1)
            cnt = jnp.maximum(cr[0, :, :1] + cr[1, :, :1], 1.0)
            t = feat / cnt
            m = t if m is None else m + t
        pn = m * inv_nrel + jnp.dot(xr[...], wr[...],
                                    preferred_element_type=jnp.float32) + br[...]
        orf[...] = jnp.maximum(pn, 0.0) + xr[...]

    nd_pad = accs[0].shape[0] // 2
    a3 = [a.reshape(2, nd_pad, 32) for a in accs]
    c3 = list(cnts)
    return pl.pallas_call(
        kern,
        grid=(g,),
        in_specs=(
            [pl.BlockSpec((2, _BN, 32), lambda i: (0, i, 0))] * nr
            + [pl.BlockSpec((2, _BN, 16), lambda i: (0, i, 0))] * nr
            + [pl.BlockSpec((_BN, _HD), lambda i: (i, 0)),
               pl.BlockSpec((_HD, _HD), lambda i: (0, 0)),
               pl.BlockSpec((1, _HD), lambda i: (0, 0))]
        ),
        out_specs=pl.BlockSpec((_BN, _HD), lambda i: (i, 0)),
        out_shape=jax.ShapeDtypeStruct((n, _HD), jnp.float32),
    )(*a3, *c3, x, w, b.reshape(1, _HD))


def _colsum(x):
    n = x.shape[0]
    g = n // _BN

    def kern(xr, orf):
        @pl.when(pl.program_id(0) == 0)
        def _():
            orf[...] = jnp.zeros_like(orf)

        orf[...] += jnp.sum(xr[...], axis=0, keepdims=True)

    return pl.pallas_call(
        kern,
        grid=(g,),
        in_specs=[pl.BlockSpec((_BN, _HD), lambda i: (i, 0))],
        out_specs=pl.BlockSpec((1, _HD), lambda i: (0, 0)),
        out_shape=jax.ShapeDtypeStruct((1, _HD), jnp.float32),
    )(x)


def _cls_head(sums, scale, w1, b1, w2, b2):
    def kern(sr, scr, w1r, b1r, w2r, b2r, oge, olg):
        ge = sr[...] * scr[...]
        oge[...] = ge
        h = jnp.maximum(
            jnp.dot(ge, w1r[...], preferred_element_type=jnp.float32) + b1r[...], 0.0)
        olg[...] = jnp.dot(h, w2r[...], preferred_element_type=jnp.float32) + b2r[...]

    return pl.pallas_call(
        kern,
        out_shape=[jax.ShapeDtypeStruct((1, 3 * _HD), jnp.float32),
                   jax.ShapeDtypeStruct((1, 50), jnp.float32)],
    )(sums, scale, w1, b1.reshape(1, _HD), w2, b2.reshape(1, 50))


def _scores(x, w1, b1, w2t, b2):
    n = x.shape[0]
    g = n // _BN

    def kern(xr, w1r, b1r, w2r, b2r, orf):
        h = jnp.maximum(
            jnp.dot(xr[...], w1r[...], preferred_element_type=jnp.float32) + b1r[...],
            0.0)
        sv = jnp.sum(h * w2r[...], axis=1, keepdims=True)
        orf[...] = sv + b2r[...]

    out = pl.pallas_call(
        kern,
        grid=(g,),
        in_specs=[pl.BlockSpec((_BN, _HD), lambda i: (i, 0)),
                  pl.BlockSpec((_HD, 32), lambda i: (0, 0)),
                  pl.BlockSpec((1, 32), lambda i: (0, 0)),
                  pl.BlockSpec((1, 32), lambda i: (0, 0)),
                  pl.BlockSpec((1, 128), lambda i: (0, 0))],
        out_specs=pl.BlockSpec((_BN, 128), lambda i: (i, 0)),
        out_shape=jax.ShapeDtypeStruct((n, 128), jnp.float32),
    )(x, w1, b1.reshape(1, 32), w2t, b2)
    return out[:, 0]


def kernel(person_x, object_x, location_x, acts_edge, uses_src, uses_dst, at_src,
           at_dst, person_table, object_table, location_table, conv_Wn, conv_Wr,
           conv_b, cls_W1, cls_b1, cls_W2, cls_b2, sp_W1, sp_b1, sp_W2, sp_b2):
    f32 = jnp.float32
    xp = _embed(person_table, person_x, _NP)
    xo = _embed(object_table, object_x, _NO)
    xl = _embed(location_table, location_x, _NL)

    e0 = _pad_edges(acts_edge[0], acts_edge[1], _NP, _NP)
    e1 = _pad_edges(uses_src, uses_dst, _NP, _NO)
    e2 = _pad_edges(uses_dst, uses_src, _NO, _NP)
    e3 = _pad_edges(at_src, at_dst, _NP, _NL)
    e4 = _pad_edges(at_dst, at_src, _NL, _NP)

    c0, c1, c2, c3, c4 = _counts((e0, e1, e2, e3, e4))

    wr_p = (conv_Wr[:, 0] + conv_Wr[:, 2] + conv_Wr[:, 4]) / 3.0
    b_p = (conv_b[:, 0] + conv_b[:, 2] + conv_b[:, 4]) / 3.0

    for l in range(3):
        z0, z1, z3 = _mm_split(xp, [conv_Wn[l, 0], conv_Wn[l, 1], conv_Wn[l, 3]])
        (z2,) = _mm_split(xo, [conv_Wn[l, 2]])
        (z4,) = _mm_split(xl, [conv_Wn[l, 4]])
        s0 = _seg(z0.reshape(2 * _NP, 32), *e0)
        s1 = _seg(z1.reshape(2 * _NP, 32), *e1)
        s2 = _seg(z2.reshape(2 * _NO, 32), *e2)
        s3 = _seg(z3.reshape(2 * _NP, 32), *e3)
        s4 = _seg(z4.reshape(2 * _NL, 32), *e4)
        xp = _combine([s0, s2, s4], [c0, c2, c4], xp, wr_p[l], b_p[l], 1.0 / 3.0)
        xo = _combine([s1], [c1], xo, conv_Wr[l, 1], conv_b[l, 1], 1.0)
        xl = _combine([s3], [c3], xl, conv_Wr[l, 3], conv_b[l, 3], 1.0)

    sums = jnp.concatenate([_colsum(xp), _colsum(xo), _colsum(xl)], axis=1)
    scale = jnp.concatenate(
        [jnp.full((1, _HD), 1.0 / _NP, f32), jnp.full((1, _HD), 1.0 / _NO, f32),
         jnp.full((1, _HD), 1.0 / _NL, f32)], axis=1)
    ge, logits = _cls_head(sums, scale, cls_W1, cls_b1, cls_W2, cls_b2)

    b2full = jnp.full((1, 128), sp_b2[0], f32)
    scores = _scores(xp, sp_W1, sp_b1, sp_W2.reshape(1, 32), b2full)

    return logits, scores, ge.reshape(3 * _HD)


# speedup vs baseline: 10.5175x; 1.0264x over previous
import functools

import jax
import jax.numpy as jnp
from jax import lax
from jax.experimental import pallas as pl
from jax.experimental.pallas import tpu as pltpu
from jax.experimental.pallas import tpu_sc as plsc

_HD = 64
_NP, _NO, _NL = 50000, 20000, 10000
_E = 800000
_NC = 2
_NS = 16
_K = 4
_KBIG = 14
_PT = 50176
_EPAD = _NS * _PT + 2 * _KBIG * 128
_ROWS_PER_TILE = _PT // 128
_NMACRO = _ROWS_PER_TILE // _K


def _ndpad(n):
    return -(-(n + 16) // 256) * 256


@functools.lru_cache(None)
def _seg_kernel(n_src, nd_pad):
    zr = nd_pad // 16
    k = _K if nd_pad > 30000 else _KBIG
    nmacro = _ROWS_PER_TILE // k
    mesh = plsc.VectorSubcoreMesh(core_axis_name="c", subcore_axis_name="s")

    @functools.partial(
        pl.kernel,
        mesh=mesh,
        compiler_params=pltpu.CompilerParams(use_tc_tiling_on_sc=False),
        out_type=jax.ShapeDtypeStruct((2 * nd_pad, 32), jnp.float32),
        scratch_types=[
            pltpu.VMEM((k, 128), jnp.int32),
            pltpu.VMEM((k, 128), jnp.int32),
            pltpu.VMEM((k, 128), jnp.int32),
            pltpu.VMEM((k, 128), jnp.int32),
            pltpu.VMEM((k * 128, 32), jnp.float32),
            pltpu.VMEM_SHARED((nd_pad, 32), jnp.float32),
            pltpu.SemaphoreType.DMA,
            pltpu.SemaphoreType.DMA,
            pltpu.SemaphoreType.DMA,
        ],
    )
    def seg(z_hbm, src_hbm, dst_hbm, zero_hbm, out_hbm,
            src_a, dst_a, src_b, dst_b, rows, acc, sem, sem_idx, sem_s):
        c = lax.axis_index("c")
        s = lax.axis_index("s")
        off = c * n_src
        r0 = s * zr
        pltpu.sync_copy(zero_hbm.at[pl.ds(r0, zr)], acc.at[pl.ds(r0, zr)])
        plsc.subcore_barrier()

        def fire_idx(m, srcv, dstv):
            rb = s * _ROWS_PER_TILE + m * k
            pltpu.async_copy(src_hbm.at[pl.ds(rb, k)], srcv, sem_idx)
            pltpu.async_copy(dst_hbm.at[pl.ds(rb, k)], dstv, sem_idx)

        def wait_idx(srcv, dstv):
            pltpu.make_async_copy(src_hbm.at[pl.ds(0, k)], srcv, sem_idx).wait()
            pltpu.make_async_copy(dst_hbm.at[pl.ds(0, k)], dstv, sem_idx).wait()

        def adjust(srcv):
            for j in range(k):
                for t in range(8):
                    srcv[j, pl.ds(t * 16, 16)] = srcv[j, pl.ds(t * 16, 16)] + off

        def fire(srcv, j):
            pltpu.async_copy(z_hbm.at[srcv.at[j]], rows.at[pl.ds(j * 128, 128)], sem)

        def drain(j):
            pltpu.make_async_copy(
                z_hbm.at[pl.ds(0, 128)], rows.at[pl.ds(j * 128, 128)], sem).wait()

        def chunk(m, cur_s, cur_d, nxt_s, nxt_d):
            wait_idx(nxt_s, nxt_d)
            adjust(nxt_s)
            for j in range(k):
                drain(j)
            for j in range(k):
                pltpu.async_copy(rows.at[pl.ds(j * 128, 128)],
                                 acc.at[cur_d.at[j]], sem_s, add=True)
            for j in range(k):
                pltpu.make_async_copy(rows.at[pl.ds(j * 128, 128)],
                                      acc.at[pl.ds(0, 128)], sem_s).wait()
                fire(nxt_s, j)
            fire_idx(m + 2, cur_s, cur_d)

        pltpu.sync_copy(src_hbm.at[pl.ds(s * _ROWS_PER_TILE, k)], src_a)
        pltpu.sync_copy(dst_hbm.at[pl.ds(s * _ROWS_PER_TILE, k)], dst_a)
        adjust(src_a)
        for j in range(k):
            fire(src_a, j)
        fire_idx(1, src_b, dst_b)

        def body(i, carry):
            chunk(2 * i, src_a, dst_a, src_b, dst_b)
            chunk(2 * i + 1, src_b, dst_b, src_a, dst_a)
            return carry

        lax.fori_loop(0, nmacro // 2, body, 0)
        wait_idx(src_b, dst_b)
        for j in range(k):
            drain(j)
        plsc.subcore_barrier()
        pltpu.sync_copy(acc.at[pl.ds(r0, zr)], out_hbm.at[pl.ds(c * nd_pad + r0, zr)])

    return seg


def _seg(z2, src_rows, dst_rows, n_src, n_dst):
    nd_pad = _ndpad(n_dst)
    zeros = jnp.zeros((nd_pad, 32), jnp.float32)
    return _seg_kernel(n_src, nd_pad)(z2, src_rows, dst_rows, zeros)


_CNT_NDS = (_ndpad(_NP), _ndpad(_NO), _ndpad(_NP), _ndpad(_NL), _ndpad(_NP))


@functools.lru_cache(None)
def _counts_kernel():
    mesh = plsc.VectorSubcoreMesh(core_axis_name="c", subcore_axis_name="s")
    half = _ROWS_PER_TILE // 2
    kc = _KBIG
    nch = half // kc
    nd_max = max(_CNT_NDS)

    @functools.partial(
        pl.kernel,
        mesh=mesh,
        compiler_params=pltpu.CompilerParams(use_tc_tiling_on_sc=False),
        out_type=[jax.ShapeDtypeStruct((2 * nd, 16), jnp.float32)
                  for nd in _CNT_NDS],
        scratch_types=[
            pltpu.VMEM((kc, 128), jnp.int32),
            pltpu.VMEM((kc, 128), jnp.int32),
            pltpu.VMEM((128, 16), jnp.float32),
            pltpu.VMEM_SHARED((nd_max, 16), jnp.float32),
            pltpu.SemaphoreType.DMA,
            pltpu.SemaphoreType.DMA,
        ],
    )
    def cnt(d0, d1, d2, d3, d4, zero_hbm, o0, o1, o2, o3, o4, dst_a, dst_b,
            ones, cacc, sem_s, sem_idx):
        c = lax.axis_index("c")
        s = lax.axis_index("s")
        for i in range(128):
            ones[i, pl.ds(0, 16)] = jnp.full((16,), 1.0, jnp.float32)
        for dh, oh, nd in zip((d0, d1, d2, d3, d4), (o0, o1, o2, o3, o4),
                              _CNT_NDS):
            zrr = nd // 16
            r0 = s * zrr
            base = s * _ROWS_PER_TILE + c * half
            pltpu.sync_copy(zero_hbm.at[pl.ds(r0, zrr)], cacc.at[pl.ds(r0, zrr)])
            plsc.subcore_barrier()
            pltpu.sync_copy(dh.at[pl.ds(base, kc)], dst_a)
            pltpu.async_copy(dh.at[pl.ds(base + kc, kc)], dst_b, sem_idx)

            def chunk(m, cur_d, nxt_d, dh=dh, base=base):
                pltpu.make_async_copy(dh.at[pl.ds(0, kc)], nxt_d,
                                      sem_idx).wait()
                for j in range(kc):
                    pltpu.async_copy(ones, cacc.at[cur_d.at[j]], sem_s,
                                     add=True)
                for j in range(kc):
                    pltpu.make_async_copy(ones, cacc.at[pl.ds(0, 128)],
                                          sem_s).wait()
                pltpu.async_copy(dh.at[pl.ds(base + (m + 2) * kc, kc)],
                                 cur_d, sem_idx)

            def body(i, carry, chunk=chunk):
                chunk(2 * i, dst_a, dst_b)
                chunk(2 * i + 1, dst_b, dst_a)
                return carry

            lax.fori_loop(0, nch // 2, body, 0)
            pltpu.make_async_copy(dh.at[pl.ds(0, kc)], dst_b, sem_idx).wait()
            plsc.subcore_barrier()
            pltpu.sync_copy(cacc.at[pl.ds(r0, zrr)],
                            oh.at[pl.ds(c * nd + r0, zrr)])
            plsc.subcore_barrier()

    return cnt


def _counts(edges):
    zeros = jnp.zeros((max(_CNT_NDS), 16), jnp.float32)
    outs = _counts_kernel()(*[e[1] for e in edges], zeros)
    return [o.reshape(2, nd, 16) for o, nd in zip(outs, _CNT_NDS)]


@functools.lru_cache(None)
def _embed_kernel(b_pad, wb, wk):
    mesh = plsc.VectorSubcoreMesh(core_axis_name="c", subcore_axis_name="s")

    @functools.partial(
        pl.kernel,
        mesh=mesh,
        compiler_params=pltpu.CompilerParams(use_tc_tiling_on_sc=False),
        out_type=jax.ShapeDtypeStruct((b_pad, _HD), jnp.float32),
        scratch_types=[
            pltpu.VMEM((wk, 128), jnp.int32),
            pltpu.VMEM((wb, _HD), jnp.float32),
            pltpu.SemaphoreType.DMA,
        ],
    )
    def emb(tab_hbm, idx_hbm, out_hbm, idxv, rows, sem):
        c = lax.axis_index("c")
        s = lax.axis_index("s")
        w = s * _NC + c
        pltpu.sync_copy(idx_hbm.at[w], idxv)
        cops = [
            pltpu.async_copy(tab_hbm.at[idxv.at[j]], rows.at[pl.ds(j * 128, 128)], sem)
            for j in range(wk)
        ]
        for cp in cops:
            cp.wait()
        pltpu.sync_copy(rows, out_hbm.at[pl.ds(w * wb, wb)])

    return emb


def _embed(table, idx, n):
    nw = _NC * _NS
    wb = -(-n // (nw * 128)) * 128
    b_pad = nw * wb
    wk = wb // 128
    pad = b_pad - n
    idx_p = jnp.concatenate(
        [idx.astype(jnp.int32), jnp.arange(pad, dtype=jnp.int32) % table.shape[0]]
    ).reshape(nw, wk, 128)
    out = _embed_kernel(b_pad, wb, wk)(table, idx_p)
    return out[:n]


def _pad_edges(src, dst, n_src, n_dst):
    pad = _EPAD - _E
    ar = jnp.arange(pad, dtype=jnp.int32)
    src_p = jnp.concatenate([src.astype(jnp.int32), ar % n_src])
    dst_p = jnp.concatenate([dst.astype(jnp.int32), n_dst + (ar % 16)])
    return (src_p.reshape(_EPAD // 128, 128), dst_p.reshape(_EPAD // 128, 128),
            n_src, n_dst)


_BN = 2000


def _mm_split(x, ws):
    n = x.shape[0]
    g = n // _BN
    nw = len(ws)

    def kern(*refs):
        x_ref = refs[0]
        w_refs = refs[1:1 + nw]
        o_refs = refs[1 + nw:]
        xb = x_ref[...]
        for wr, orf in zip(w_refs, o_refs):
            z = jnp.dot(xb, wr[...], preferred_element_type=jnp.float32)
            orf[0] = z[:, :32]
            orf[1] = z[:, 32:64]

    return pl.pallas_call(
        kern,
        grid=(g,),
        in_specs=[pl.BlockSpec((_BN, _HD), lambda i: (i, 0))]
        + [pl.BlockSpec((_HD, _HD), lambda i: (0, 0))] * nw,
        out_specs=[pl.BlockSpec((2, _BN, 32), lambda i: (0, i, 0))] * nw,
        out_shape=[jax.ShapeDtypeStruct((2, n, 32), jnp.float32)] * nw,
    )(x, *ws)


def _combine(accs, cnts, x, w, b, inv_nrel):
    n = x.shape[0]
    g = n // _BN
    nr = len(accs)

    def kern(*refs):
        a_refs = refs[:nr]
        c_refs = refs[nr:2 * nr]
        xr, wr, br = refs[2 * nr:2 * nr + 3]
        orf = refs[2 * nr + 3]
        m = None
        for ar, cr in zip(a_refs, c_refs):
            feat = jnp.concatenate([ar[0], ar[1]], axis=1)
            cnt = jnp.maximum(cr[0, :, :1] + cr[1, :, :1], 1.0)
            t = feat / cnt
            m = t if m is None else m + t
        pn = m * inv_nrel + jnp.dot(xr[...], wr[...],
                                    preferred_element_type=jnp.float32) + br[...]
        orf[...] = jnp.maximum(pn, 0.0) + xr[...]

    nd_pad = accs[0].shape[0] // 2
    a3 = [a.reshape(2, nd_pad, 32) for a in accs]
    c3 = list(cnts)
    return pl.pallas_call(
        kern,
        grid=(g,),
        in_specs=(
            [pl.BlockSpec((2, _BN, 32), lambda i: (0, i, 0))] * nr
            + [pl.BlockSpec((2, _BN, 16), lambda i: (0, i, 0))] * nr
            + [pl.BlockSpec((_BN, _HD), lambda i: (i, 0)),
               pl.BlockSpec((_HD, _HD), lambda i: (0, 0)),
               pl.BlockSpec((1, _HD), lambda i: (0, 0))]
        ),
        out_specs=pl.BlockSpec((_BN, _HD), lambda i: (i, 0)),
        out_shape=jax.ShapeDtypeStruct((n, _HD), jnp.float32),
    )(*a3, *c3, x, w, b.reshape(1, _HD))


def _colsum(x):
    n = x.shape[0]
    g = n // _BN

    def kern(xr, orf):
        @pl.when(pl.program_id(0) == 0)
        def _():
            orf[...] = jnp.zeros_like(orf)

        orf[...] += jnp.sum(xr[...], axis=0, keepdims=True)

    return pl.pallas_call(
        kern,
        grid=(g,),
        in_specs=[pl.BlockSpec((_BN, _HD), lambda i: (i, 0))],
        out_specs=pl.BlockSpec((1, _HD), lambda i: (0, 0)),
        out_shape=jax.ShapeDtypeStruct((1, _HD), jnp.float32),
    )(x)


def _cls_head(sums, scale, w1, b1, w2, b2):
    def kern(sr, scr, w1r, b1r, w2r, b2r, oge, olg):
        ge = sr[...] * scr[...]
        oge[...] = ge
        h = jnp.maximum(
            jnp.dot(ge, w1r[...], preferred_element_type=jnp.float32) + b1r[...], 0.0)
        olg[...] = jnp.dot(h, w2r[...], preferred_element_type=jnp.float32) + b2r[...]

    return pl.pallas_call(
        kern,
        out_shape=[jax.ShapeDtypeStruct((1, 3 * _HD), jnp.float32),
                   jax.ShapeDtypeStruct((1, 50), jnp.float32)],
    )(sums, scale, w1, b1.reshape(1, _HD), w2, b2.reshape(1, 50))


def _scores(x, w1, b1, w2t, b2):
    n = x.shape[0]
    g = n // _BN

    def kern(xr, w1r, b1r, w2r, b2r, orf):
        h = jnp.maximum(
            jnp.dot(xr[...], w1r[...], preferred_element_type=jnp.float32) + b1r[...],
            0.0)
        sv = jnp.sum(h * w2r[...], axis=1, keepdims=True)
        orf[...] = sv + b2r[...]

    out = pl.pallas_call(
        kern,
        grid=(g,),
        in_specs=[pl.BlockSpec((_BN, _HD), lambda i: (i, 0)),
                  pl.BlockSpec((_HD, 32), lambda i: (0, 0)),
                  pl.BlockSpec((1, 32), lambda i: (0, 0)),
                  pl.BlockSpec((1, 32), lambda i: (0, 0)),
                  pl.BlockSpec((1, 128), lambda i: (0, 0))],
        out_specs=pl.BlockSpec((_BN, 128), lambda i: (i, 0)),
        out_shape=jax.ShapeDtypeStruct((n, 128), jnp.float32),
    )(x, w1, b1.reshape(1, 32), w2t, b2)
    return out[:, 0]


def kernel(person_x, object_x, location_x, acts_edge, uses_src, uses_dst, at_src,
           at_dst, person_table, object_table, location_table, conv_Wn, conv_Wr,
           conv_b, cls_W1, cls_b1, cls_W2, cls_b2, sp_W1, sp_b1, sp_W2, sp_b2):
    f32 = jnp.float32
    xp = _embed(person_table, person_x, _NP)
    xo = _embed(object_table, object_x, _NO)
    xl = _embed(location_table, location_x, _NL)

    e0 = _pad_edges(acts_edge[0], acts_edge[1], _NP, _NP)
    e1 = _pad_edges(uses_src, uses_dst, _NP, _NO)
    e2 = _pad_edges(uses_dst, uses_src, _NO, _NP)
    e3 = _pad_edges(at_src, at_dst, _NP, _NL)
    e4 = _pad_edges(at_dst, at_src, _NL, _NP)

    c0, c1, c2, c3, c4 = _counts((e0, e1, e2, e3, e4))

    wr_p = (conv_Wr[:, 0] + conv_Wr[:, 2] + conv_Wr[:, 4]) / 3.0
    b_p = (conv_b[:, 0] + conv_b[:, 2] + conv_b[:, 4]) / 3.0

    for l in range(3):
        z0, z1, z3 = _mm_split(xp, [conv_Wn[l, 0], conv_Wn[l, 1], conv_Wn[l, 3]])
        (z2,) = _mm_split(xo, [conv_Wn[l, 2]])
        (z4,) = _mm_split(xl, [conv_Wn[l, 4]])
        s0 = _seg(z0.reshape(2 * _NP, 32), *e0)
        s1 = _seg(z1.reshape(2 * _NP, 32), *e1)
        s2 = _seg(z2.reshape(2 * _NO, 32), *e2)
        s3 = _seg(z3.reshape(2 * _NP, 32), *e3)
        s4 = _seg(z4.reshape(2 * _NL, 32), *e4)
        xp = _combine([s0, s2, s4], [c0, c2, c4], xp, wr_p[l], b_p[l], 1.0 / 3.0)
        xo = _combine([s1], [c1], xo, conv_Wr[l, 1], conv_b[l, 1], 1.0)
        xl = _combine([s3], [c3], xl, conv_Wr[l, 3], conv_b[l, 3], 1.0)

    sums = jnp.concatenate([_colsum(xp), _colsum(xo), _colsum(xl)], axis=1)
    scale = jnp.concatenate(
        [jnp.full((1, _HD), 1.0 / _NP, f32), jnp.full((1, _HD), 1.0 / _NO, f32),
         jnp.full((1, _HD), 1.0 / _NL, f32)], axis=1)
    ge, logits = _cls_head(sums, scale, cls_W1, cls_b1, cls_W2, cls_b2)

    b2full = jnp.full((1, 128), sp_b2[0], f32)
    scores = _scores(xp, sp_W1, sp_b1, sp_W2.reshape(1, 32), b2full)

    return logits, scores, ge.reshape(3 * _HD)
